# baseline (speedup 1.0000x reference)
import functools

import jax
import jax.numpy as jnp
from jax import lax
from jax.experimental import pallas as pl
from jax.experimental.pallas import tpu as pltpu
from jax.experimental.pallas import tpu_sc as plsc

N = 100000
NODE_DIM = 10
EDGE_DIM = 16
EMB = 20
TIME = 10
E = 1600000

_NODE_BLK = 2000
_EDGE_BLK = 8000

_NP = 100096
_M = 2 * E
_MP = 3276800
_HOP_IPB = 128
_HOP_K = 4
_HOP_CHUNKS = (_MP // 32) // (_HOP_K * _HOP_IPB)


def _hop_body(x_ref, src_ref, dst_ref, zeros_ref, out_ref,
              idxs_v, idxd_v, rows_v, acc, sem):
    c = lax.axis_index("c")
    s = lax.axis_index("s")
    wid = s * 2 + c
    rows_per_tile = _NP // 16
    r0 = s * rows_per_tile
    pltpu.sync_copy(zeros_ref.at[pl.ds(r0, rows_per_tile)],
                    acc.at[pl.ds(r0, rows_per_tile)])
    plsc.subcore_barrier()

    idx_row0 = wid * (_HOP_CHUNKS * _HOP_K)

    def chunk(ch, carry):
        rb = idx_row0 + ch * _HOP_K
        pltpu.sync_copy(src_ref.at[pl.ds(rb, _HOP_K)], idxs_v)
        cps = [pltpu.async_copy(x_ref.at[idxs_v.at[j]],
                                rows_v.at[pl.ds(j * _HOP_IPB, _HOP_IPB)], sem)
               for j in range(_HOP_K)]
        for cp in cps:
            cp.wait()
        pltpu.sync_copy(dst_ref.at[pl.ds(rb, _HOP_K)], idxd_v)
        for j in range(_HOP_K):
            pltpu.sync_copy(rows_v.at[pl.ds(j * _HOP_IPB, _HOP_IPB)],
                            acc.at[idxd_v.at[j]], add=True)
        return carry

    lax.fori_loop(0, _HOP_CHUNKS, chunk, 0)
    plsc.subcore_barrier()
    pltpu.sync_copy(acc.at[pl.ds(r0, rows_per_tile)],
                    out_ref.at[c, pl.ds(r0, rows_per_tile)])


def _hop(x16, src2, dst2, zeros16):
    mesh = plsc.VectorSubcoreMesh(core_axis_name="c", subcore_axis_name="s")
    f = pl.kernel(
        _hop_body,
        out_type=jax.ShapeDtypeStruct((2, _NP, 16), jnp.float32),
        mesh=mesh,
        scratch_types=[
            pltpu.VMEM((_HOP_K, _HOP_IPB), jnp.int32),
            pltpu.VMEM((_HOP_K, _HOP_IPB), jnp.int32),
            pltpu.VMEM((_HOP_K * _HOP_IPB, 16), jnp.float32),
            pltpu.VMEM_SHARED((_NP, 16), jnp.float32),
            pltpu.SemaphoreType.DMA,
        ],
        compiler_params=pltpu.CompilerParams(use_tc_tiling_on_sc=False, needs_layout_passes=False),
    )
    return f(x16, src2, dst2, zeros16)


def _merge_body(a_ref, b_ref, o_ref):
    o_ref[...] = a_ref[0] + b_ref[0]


def _merge(p):
    blk = 6256
    return pl.pallas_call(
        _merge_body,
        grid=(_NP // blk,),
        in_specs=[
            pl.BlockSpec((1, blk, 16), lambda i: (0, i, 0)),
            pl.BlockSpec((1, blk, 16), lambda i: (1, i, 0)),
        ],
        out_specs=pl.BlockSpec((blk, 16), lambda i: (i, 0)),
        out_shape=jax.ShapeDtypeStruct((_NP, 16), jnp.float32),
    )(p, p)


_EP = 1605632
_AT_C = 256
_AT_CHUNKS = (_EP // 32) // _AT_C
_SM_C = 512
_SM_CHUNKS = (_EP // 32) // _SM_C
_NPT = _NP // 16
_ISQ = 0.22360679774997896


def _iota16():
    return lax.iota(jnp.int32, 16)


def _pass1_body(kv_ref, q_ref, as_ref, ad_ref, ea_ref,
                src_ref, dst_ref, srcn_ref, dstn_ref,
                msga_ref, msgb_ref, alpha_ref,
                si_v, di_v, ni_v, mi_v, kv_v, q_v, as_v, ad_v, ea_v,
                msga_v, msgb_v, al_v, sem):
    c = lax.axis_index("c")
    s = lax.axis_index("s")
    wid = s * 2 + c
    edge0 = wid * (_AT_CHUNKS * _AT_C)
    irow0 = edge0 // 128

    def chunk(ch, carry):
        base = edge0 + ch * _AT_C
        rb = irow0 + ch * (_AT_C // 128)
        pltpu.sync_copy(src_ref.at[pl.ds(rb, 2)], si_v)
        pltpu.sync_copy(dst_ref.at[pl.ds(rb, 2)], di_v)
        pltpu.sync_copy(srcn_ref.at[pl.ds(rb, 2)], ni_v)
        pltpu.sync_copy(dstn_ref.at[pl.ds(rb, 2)], mi_v)
        cps = []
        for j in range(2):
            sl = pl.ds(j * 128, 128)
            cps.append(pltpu.async_copy(kv_ref.at[si_v.at[j]], kv_v.at[sl], sem))
            cps.append(pltpu.async_copy(q_ref.at[di_v.at[j]], q_v.at[sl], sem))
            cps.append(pltpu.async_copy(as_ref.at[ni_v.at[j]], as_v.at[sl], sem))
            cps.append(pltpu.async_copy(ad_ref.at[mi_v.at[j]], ad_v.at[sl], sem))
        pltpu.sync_copy(ea_ref.at[pl.ds(base, _AT_C)], ea_v)
        for cp in cps:
            cp.wait()

        def block(g, carry2):
            i16 = _iota16() + g * 16
            acc = jnp.zeros((16,), jnp.float32)
            for d in range(EMB):
                cd = jnp.full((16,), d, jnp.int32)
                e_d = (plsc.load_gather(ea_v, [i16, cd])
                       + plsc.load_gather(as_v, [i16, cd])
                       + plsc.load_gather(ad_v, [i16, cd]))
                k_d = plsc.load_gather(kv_v, [i16, cd])
                v_d = plsc.load_gather(kv_v, [i16, jnp.full((16,), EMB + d, jnp.int32)])
                q_d = plsc.load_gather(q_v, [i16, cd])
                acc = acc + q_d * (k_d + e_d)
                if d < 16:
                    plsc.store_scatter(msga_v, [i16, cd], v_d + e_d)
                else:
                    cb = jnp.full((16,), d - 16, jnp.int32)
                    plsc.store_scatter(msgb_v, [i16, cb], v_d + e_d)
            z16 = jnp.zeros((16,), jnp.float32)
            for d in range(4, 8):
                plsc.store_scatter(msgb_v, [i16, jnp.full((16,), d, jnp.int32)], z16)
            al_v[pl.ds(g * 16, 16)] = acc * _ISQ
            return carry2

        lax.fori_loop(0, _AT_C // 16, block, 0)
        pltpu.sync_copy(msga_v, msga_ref.at[pl.ds(base, _AT_C)])
        pltpu.sync_copy(msgb_v, msgb_ref.at[pl.ds(base, _AT_C)])
        pltpu.sync_copy(al_v, alpha_ref.at[pl.ds(base, _AT_C)])
        return carry

    lax.fori_loop(0, _AT_CHUNKS, chunk, 0)


def _pass1(kv48, q32, as32, ad32, ea20, src2, dst2, srcn2, dstn2):
    mesh = plsc.VectorSubcoreMesh(core_axis_name="c", subcore_axis_name="s")
    f = pl.kernel(
        _pass1_body,
        out_type=(jax.ShapeDtypeStruct((_EP, 16), jnp.float32),
                  jax.ShapeDtypeStruct((_EP, 8), jnp.float32),
                  jax.ShapeDtypeStruct((_EP,), jnp.float32)),
        mesh=mesh,
        scratch_types=[
            pltpu.VMEM((2, 128), jnp.int32),
            pltpu.VMEM((2, 128), jnp.int32),
            pltpu.VMEM((2, 128), jnp.int32),
            pltpu.VMEM((2, 128), jnp.int32),
            pltpu.VMEM((_AT_C, 48), jnp.float32),
            pltpu.VMEM((_AT_C, 32), jnp.float32),
            pltpu.VMEM((_AT_C, 32), jnp.float32),
            pltpu.VMEM((_AT_C, 32), jnp.float32),
            pltpu.VMEM((_AT_C, EMB), jnp.float32),
            pltpu.VMEM((_AT_C, 16), jnp.float32),
            pltpu.VMEM((_AT_C, 8), jnp.float32),
            pltpu.VMEM((_AT_C,), jnp.float32),
            pltpu.SemaphoreType.DMA,
        ],
        compiler_params=pltpu.CompilerParams(use_tc_tiling_on_sc=False, needs_layout_passes=False),
    )
    return f(kv48, q32, as32, ad32, ea20, src2, dst2, srcn2, dstn2)


def _seg_max_body(alpha_ref, dst_ref, amax_ref,
                  maxtab, a_v, d_v, sem):
    c = lax.axis_index("c")
    s = lax.axis_index("s")
    wid = s * 2 + c

    def initb(i, carry):
        maxtab[pl.ds(i * 16, 16)] = jnp.full((16,), -3e38, jnp.float32)
        return carry

    lax.fori_loop(0, _NP // 16, initb, 0)

    edge0 = wid * (_SM_CHUNKS * _SM_C)
    irow0 = edge0 // 128

    def chunk(ch, carry):
        base = edge0 + ch * _SM_C
        rb = irow0 + ch * (_SM_C // 128)
        pltpu.sync_copy(alpha_ref.at[pl.ds(base, _SM_C)], a_v)
        pltpu.sync_copy(dst_ref.at[pl.ds(rb, _SM_C // 128)], d_v)

        def block(g, carry2):
            a16 = a_v[pl.ds(g * 16, 16)]
            j = g // 8
            o = (g % 8) * 16
            d16 = d_v[j, pl.ds(o, 16)]
            m0 = jnp.maximum(plsc.load_gather(maxtab, [d16]), a16)

            def cond(st):
                return st[1]

            def bodyw(st):
                m, _ = st
                plsc.store_scatter(maxtab, [d16], m)
                cur = plsc.load_gather(maxtab, [d16])
                return (jnp.maximum(cur, m), jnp.any(cur < m))

            lax.while_loop(cond, bodyw, (m0, True))
            return carry2

        lax.fori_loop(0, _SM_C // 16, block, 0)
        return carry

    lax.fori_loop(0, _SM_CHUNKS, chunk, 0)
    pltpu.sync_copy(maxtab, amax_ref.at[wid])


def _seg_max(alpha, dst2):
    mesh = plsc.VectorSubcoreMesh(core_axis_name="c", subcore_axis_name="s")
    f = pl.kernel(
        _seg_max_body,
        out_type=jax.ShapeDtypeStruct((32, _NP), jnp.float32),
        mesh=mesh,
        scratch_types=[
            pltpu.VMEM((_NP,), jnp.float32),
            pltpu.VMEM((_SM_C,), jnp.float32),
            pltpu.VMEM((_SM_C // 128, 128), jnp.int32),
            pltpu.SemaphoreType.DMA,
        ],
        compiler_params=pltpu.CompilerParams(use_tc_tiling_on_sc=False, needs_layout_passes=False),
    )
    return f(alpha, dst2)


def _amax_merge_body(p_ref, o_ref):
    o_ref[...] = jnp.max(p_ref[...], axis=0)


def _amax_merge(amax32):
    return pl.pallas_call(
        _amax_merge_body,
        out_shape=jax.ShapeDtypeStruct((_NP,), jnp.float32),
    )(amax32)


def _exp_den_body(alpha_ref, dst_ref, amax_ref, zeros_ref, al_ref, den_ref,
                  maxtab, a_v, d_v, al_v, den_sh, sem):
    c = lax.axis_index("c")
    s = lax.axis_index("s")
    wid = s * 2 + c
    r0 = s * _NPT
    pltpu.sync_copy(zeros_ref.at[pl.ds(r0, _NPT)], den_sh.at[pl.ds(r0, _NPT)])
    pltpu.sync_copy(amax_ref, maxtab)
    plsc.subcore_barrier()

    edge0 = wid * (_SM_CHUNKS * _SM_C)
    irow0 = edge0 // 128

    def chunk(ch, carry):
        base = edge0 + ch * _SM_C
        rb = irow0 + ch * (_SM_C // 128)
        pltpu.sync_copy(alpha_ref.at[pl.ds(base, _SM_C)], a_v)
        pltpu.sync_copy(dst_ref.at[pl.ds(rb, _SM_C // 128)], d_v)

        def block(g, carry2):
            a16 = a_v[pl.ds(g * 16, 16)]
            j = g // 8
            o = (g % 8) * 16
            d16 = d_v[j, pl.ds(o, 16)]
            mx16 = plsc.load_gather(maxtab, [d16])
            al_v[pl.ds(g * 16, 16)] = jnp.exp(a16 - mx16)
            return carry2

        lax.fori_loop(0, _SM_C // 16, block, 0)
        pltpu.sync_copy(al_v, al_ref.at[pl.ds(base, _SM_C)])
        for j in range(_SM_C // 128):
            pltpu.sync_copy(al_v.at[pl.ds(j * 128, 128)],
                            den_sh.at[d_v.at[j]], add=True)
        return carry

    lax.fori_loop(0, _SM_CHUNKS, chunk, 0)
    plsc.subcore_barrier()
    pltpu.sync_copy(den_sh.at[pl.ds(r0, _NPT)], den_ref.at[c, pl.ds(r0, _NPT)])


def _exp_den(alpha, dst2, amax, zerosN):
    mesh = plsc.VectorSubcoreMesh(core_axis_name="c", subcore_axis_name="s")
    f = pl.kernel(
        _exp_den_body,
        out_type=(jax.ShapeDtypeStruct((_EP,), jnp.float32),
                  jax.ShapeDtypeStruct((2, _NP), jnp.float32)),
        mesh=mesh,
        scratch_types=[
            pltpu.VMEM((_NP,), jnp.float32),
            pltpu.VMEM((_SM_C,), jnp.float32),
            pltpu.VMEM((_SM_C // 128, 128), jnp.int32),
            pltpu.VMEM((_SM_C,), jnp.float32),
            pltpu.VMEM_SHARED((_NP,), jnp.float32),
            pltpu.SemaphoreType.DMA,
        ],
        compiler_params=pltpu.CompilerParams(use_tc_tiling_on_sc=False, needs_layout_passes=False),
    )
    return f(alpha, dst2, amax, zerosN)


def _scatter_msg_body(w, msg_ref, al_ref, dst_ref, zeros_ref, out_ref,
                      m_v, al_v, d_v, w_v, acc, sem):
    c = lax.axis_index("c")
    s = lax.axis_index("s")
    wid = s * 2 + c
    r0 = s * _NPT
    pltpu.sync_copy(zeros_ref.at[pl.ds(r0, _NPT)], acc.at[pl.ds(r0, _NPT)])
    plsc.subcore_barrier()

    edge0 = wid * (_AT_CHUNKS * _AT_C)
    irow0 = edge0 // 128

    def chunk(ch, carry):
        base = edge0 + ch * _AT_C
        rb = irow0 + ch * (_AT_C // 128)
        pltpu.sync_copy(msg_ref.at[pl.ds(base, _AT_C)], m_v)
        pltpu.sync_copy(al_ref.at[pl.ds(base, _AT_C)], al_v)
        pltpu.sync_copy(dst_ref.at[pl.ds(rb, _AT_C // 128)], d_v)

        def block(g, carry2):
            i16 = _iota16() + g * 16
            w16 = al_v[pl.ds(g * 16, 16)]
            for d in range(w):
                cd = jnp.full((16,), d, jnp.int32)
                plsc.store_scatter(w_v, [i16, cd],
                                   plsc.load_gather(m_v, [i16, cd]) * w16)
            return carry2

        lax.fori_loop(0, _AT_C // 16, block, 0)
        for j in range(_AT_C // 128):
            pltpu.sync_copy(w_v.at[pl.ds(j * 128, 128)],
                            acc.at[d_v.at[j]], add=True)
        return carry

    lax.fori_loop(0, _AT_CHUNKS, chunk, 0)
    plsc.subcore_barrier()
    pltpu.sync_copy(acc.at[pl.ds(r0, _NPT)], out_ref.at[c, pl.ds(r0, _NPT)])


def _scatter_msg(msg, al, dst2, zerosNW, w):
    mesh = plsc.VectorSubcoreMesh(core_axis_name="c", subcore_axis_name="s")
    f = pl.kernel(
        functools.partial(_scatter_msg_body, w),
        out_type=jax.ShapeDtypeStruct((2, _NP, w), jnp.float32),
        mesh=mesh,
        scratch_types=[
            pltpu.VMEM((_AT_C, w), jnp.float32),
            pltpu.VMEM((_AT_C,), jnp.float32),
            pltpu.VMEM((_AT_C // 128, 128), jnp.int32),
            pltpu.VMEM((_AT_C, w), jnp.float32),
            pltpu.VMEM_SHARED((_NP, w), jnp.float32),
            pltpu.SemaphoreType.DMA,
        ],
        compiler_params=pltpu.CompilerParams(use_tc_tiling_on_sc=False, needs_layout_passes=False),
    )
    return f(msg, al, dst2, zerosNW)


def _final_body(pa_ref, pb_ref, den_ref, skip_ref, o_ref):
    a = jnp.concatenate(
        [pa_ref[0] + pa_ref[1], (pb_ref[0] + pb_ref[1])[:, :4]], axis=-1)
    dn = den_ref[:, 0] + den_ref[:, 1]
    safe = jnp.where(dn != 0.0, dn, 1.0)[:, None]
    o_ref[...] = jnp.where(dn[:, None] != 0.0, a / safe, 0.0) + skip_ref[...]


def _final(out_pa, out_pb, den_t, skip):
    blk = 2000
    return pl.pallas_call(
        _final_body,
        grid=(N // blk,),
        in_specs=[
            pl.BlockSpec((2, blk, 16), lambda i: (0, i, 0)),
            pl.BlockSpec((2, blk, 8), lambda i: (0, i, 0)),
            pl.BlockSpec((blk, 2), lambda i: (i, 0)),
            pl.BlockSpec((blk, EMB), lambda i: (i, 0)),
        ],
        out_specs=pl.BlockSpec((blk, EMB), lambda i: (i, 0)),
        out_shape=jax.ShapeDtypeStruct((N, EMB), jnp.float32),
    )(out_pa, out_pb, den_t, skip)


_NP_BLK = 3128


def _node_prep_body(x3f_ref, enc_ref, z_ref, W3T_ref, W1Tn_ref, W4T_ref,
                    W2Tn_ref, WqT_ref, WkT_ref, WvT_ref, WsT_ref, b_ref,
                    kv_ref, q_ref, as_ref, ad_ref, skip_ref):
    x3 = x3f_ref[:, :NODE_DIM]
    enc = enc_ref[...]
    z = z_ref[...]
    dot = functools.partial(jnp.dot, preferred_element_type=jnp.float32)
    As = dot(x3, W3T_ref[...]) + dot(enc, W1Tn_ref[...])
    Ad = dot(x3, W4T_ref[...]) + dot(enc, W2Tn_ref[...])
    q = dot(z, WqT_ref[...]) + b_ref[0:1, :]
    k = dot(z, WkT_ref[...]) + b_ref[1:2, :]
    v = dot(z, WvT_ref[...]) + b_ref[2:3, :]
    skip = dot(z, WsT_ref[...]) + b_ref[3:4, :]
    zpad = jnp.zeros((_NP_BLK, 12), jnp.float32)
    kv_ref[...] = jnp.concatenate([k, v, zpad[:, :8]], axis=-1)
    q_ref[...] = jnp.concatenate([q, zpad], axis=-1)
    as_ref[...] = jnp.concatenate([As, zpad], axis=-1)
    ad_ref[...] = jnp.concatenate([Ad, zpad], axis=-1)
    skip_ref[...] = skip


def _node_prep(x3f, encp, zp, W3T, W1Tn, W4T, W2Tn, WqT, WkT, WvT, WsT, b4):
    grid = (_NP // _NP_BLK,)
    w20 = lambda i: (0, 0)
    out = pl.pallas_call(
        _node_prep_body,
        grid=grid,
        in_specs=[
            pl.BlockSpec((_NP_BLK, 16), lambda i: (i, 0)),
            pl.BlockSpec((_NP_BLK, TIME), lambda i: (i, 0)),
            pl.BlockSpec((_NP_BLK, EMB), lambda i: (i, 0)),
            pl.BlockSpec((NODE_DIM, EMB), w20),
            pl.BlockSpec((TIME, EMB), w20),
            pl.BlockSpec((NODE_DIM, EMB), w20),
            pl.BlockSpec((TIME, EMB), w20),
            pl.BlockSpec((EMB, EMB), w20),
            pl.BlockSpec((EMB, EMB), w20),
            pl.BlockSpec((EMB, EMB), w20),
            pl.BlockSpec((EMB, EMB), w20),
            pl.BlockSpec((4, EMB), w20),
        ],
        out_specs=[
            pl.BlockSpec((_NP_BLK, 48), lambda i: (i, 0)),
            pl.BlockSpec((_NP_BLK, 32), lambda i: (i, 0)),
            pl.BlockSpec((_NP_BLK, 32), lambda i: (i, 0)),
            pl.BlockSpec((_NP_BLK, 32), lambda i: (i, 0)),
            pl.BlockSpec((_NP_BLK, EMB), lambda i: (i, 0)),
        ],
        out_shape=[
            jax.ShapeDtypeStruct((_NP, 48), jnp.float32),
            jax.ShapeDtypeStruct((_NP, 32), jnp.float32),
            jax.ShapeDtypeStruct((_NP, 32), jnp.float32),
            jax.ShapeDtypeStruct((_NP, 32), jnp.float32),
            jax.ShapeDtypeStruct((_NP, EMB), jnp.float32),
        ],
    )(x3f, encp, zp, W3T, W1Tn, W4T, W2Tn, WqT, WkT, WvT, WsT, b4)
    return out


def _edge_prep_body(ea_ref, t_ref, W0T_ref, s12_ref, out_ref):
    ea = ea_ref[...]
    t = t_ref[...]
    out_ref[...] = (
        jnp.dot(ea, W0T_ref[...], preferred_element_type=jnp.float32)
        + t * s12_ref[...]
    )


def _edge_prep(edge_attr_p, t_p, W0T, s12):
    blk = 8192
    grid = (_EP // blk,)
    out = pl.pallas_call(
        _edge_prep_body,
        grid=grid,
        in_specs=[
            pl.BlockSpec((blk, EDGE_DIM), lambda i: (i, 0)),
            pl.BlockSpec((blk, 1), lambda i: (i, 0)),
            pl.BlockSpec((EDGE_DIM, EMB), lambda i: (0, 0)),
            pl.BlockSpec((1, EMB), lambda i: (0, 0)),
        ],
        out_specs=pl.BlockSpec((blk, EMB), lambda i: (i, 0)),
        out_shape=jax.ShapeDtypeStruct((_EP, EMB), jnp.float32),
    )(edge_attr_p, t_p, W0T, s12)
    return out


def kernel(x, n_id, src_n_id, dst_n_id, edge_index, edge_attr, t,
           his_edge_index, enc_t_table, z,
           Wq, bq, Wk, bk, Wv, bv, We, Ws, bs):
    pad_idx = jnp.full((_MP - _M,), N, dtype=jnp.int32)
    src2 = jnp.concatenate(
        [edge_index[0], his_edge_index[0], pad_idx]).reshape(_MP // 128, 128)
    dst2 = jnp.concatenate(
        [edge_index[1], his_edge_index[1], pad_idx]).reshape(_MP // 128, 128)
    zeros16 = jnp.zeros((_NP, 16), jnp.float32)
    x16 = jnp.pad(x, ((0, _NP - N), (0, 16 - NODE_DIM)))
    for _ in range(3):
        x16 = _merge(_hop(x16, src2, dst2, zeros16))

    W0 = We[:, :EDGE_DIM]
    W1 = We[:, EDGE_DIM:EDGE_DIM + TIME]
    W2 = We[:, EDGE_DIM + TIME:EDGE_DIM + 2 * TIME]
    W3 = We[:, EDGE_DIM + 2 * TIME:EDGE_DIM + 2 * TIME + NODE_DIM]
    W4 = We[:, EDGE_DIM + 2 * TIME + NODE_DIM:]
    s12 = jnp.sum(W1 + W2, axis=1)[None, :]
    b4 = jnp.stack([bq, bk, bv, bs], axis=0)

    encp = jnp.pad(enc_t_table, ((0, _NP - N), (0, 0)))
    zp = jnp.pad(z, ((0, _NP - N), (0, 0)))
    kv48, q32, as32, ad32, skip = _node_prep(
        x16, encp, zp, W3.T, -W1.T, W4.T, -W2.T, Wq.T, Wk.T, Wv.T, Ws.T, b4)

    epad = _EP - E
    ea_p = jnp.pad(edge_attr, ((0, epad), (0, 0)))
    t_p = jnp.pad(t, (0, epad))[:, None]
    EA20 = _edge_prep(ea_p, t_p, W0.T, s12)

    e_pad_idx = jnp.full((epad,), N, dtype=jnp.int32)
    srcA = jnp.concatenate([edge_index[0], e_pad_idx]).reshape(_EP // 128, 128)
    dstA = jnp.concatenate([edge_index[1], e_pad_idx]).reshape(_EP // 128, 128)
    srcnA = jnp.concatenate([src_n_id, e_pad_idx]).reshape(_EP // 128, 128)
    dstnA = jnp.concatenate([dst_n_id, e_pad_idx]).reshape(_EP // 128, 128)

    src, dst = edge_index[0], edge_index[1]
    q20, k20, v20 = q32[:N, :EMB], kv48[:N, :EMB], kv48[:N, EMB:2 * EMB]
    e = EA20[:E] + as32[src_n_id, :EMB] + ad32[dst_n_id, :EMB]
    alpha_x = jnp.sum(q20[dst] * (k20[src] + e), axis=-1) / jnp.sqrt(float(EMB))
    amax_x = jax.ops.segment_max(alpha_x, dst, num_segments=N)
    amax_x = jnp.where(jnp.isfinite(amax_x), amax_x, 0.0)
    al_x = jnp.exp(alpha_x - amax_x[dst])
    den_x = jax.ops.segment_sum(al_x, dst, num_segments=N)
    msg_x = (v20[src] + e) * al_x[:, None]
    out_pre = jax.ops.segment_sum(msg_x, dst, num_segments=N)
    return (jnp.where(den_x[:, None] != 0.0, out_pre / den_x[:, None], 0.0)
            + skip[:N])

    msgA, msgB, alpha = _pass1(kv48, q32, as32, ad32, EA20,
                               srcA, dstA, srcnA, dstnA)
    amax = _amax_merge(_seg_max(alpha, dstA))
    zerosN = jnp.zeros((_NP,), jnp.float32)
    al, den_p = _exp_den(alpha, dstA, amax, zerosN)
    out_pa = _scatter_msg(msgA, al, dstA, zeros16, 16)
    zerosN8 = jnp.zeros((_NP, 8), jnp.float32)
    out_pb = _scatter_msg(msgB, al, dstA, zerosN8, 8)
    return _final(out_pa, out_pb, den_p.T, skip[:N])


# speedup vs baseline: 9.8790x; 9.8790x over previous
import functools

import jax
import jax.numpy as jnp
from jax import lax
from jax.experimental import pallas as pl
from jax.experimental.pallas import tpu as pltpu
from jax.experimental.pallas import tpu_sc as plsc

N = 100000
NODE_DIM = 10
EDGE_DIM = 16
EMB = 20
TIME = 10
E = 1600000

_NODE_BLK = 2000
_EDGE_BLK = 8000

_NP = 100096
_M = 2 * E
_MP = 3276800
_HOP_IPB = 128
_HOP_K = 4
_HOP_CHUNKS = (_MP // 32) // (_HOP_K * _HOP_IPB)


def _hop_body(x_ref, src_ref, dst_ref, zeros_ref, out_ref,
              idxs_v, idxd_v, rows_v, acc, sem):
    c = lax.axis_index("c")
    s = lax.axis_index("s")
    wid = s * 2 + c
    rows_per_tile = _NP // 16
    r0 = s * rows_per_tile
    pltpu.sync_copy(zeros_ref.at[pl.ds(r0, rows_per_tile)],
                    acc.at[pl.ds(r0, rows_per_tile)])
    plsc.subcore_barrier()

    idx_row0 = wid * (_HOP_CHUNKS * _HOP_K)

    def chunk(ch, carry):
        rb = idx_row0 + ch * _HOP_K
        pltpu.sync_copy(src_ref.at[pl.ds(rb, _HOP_K)], idxs_v)
        cps = [pltpu.async_copy(x_ref.at[idxs_v.at[j]],
                                rows_v.at[pl.ds(j * _HOP_IPB, _HOP_IPB)], sem)
               for j in range(_HOP_K)]
        for cp in cps:
            cp.wait()
        pltpu.sync_copy(dst_ref.at[pl.ds(rb, _HOP_K)], idxd_v)
        for j in range(_HOP_K):
            pltpu.sync_copy(rows_v.at[pl.ds(j * _HOP_IPB, _HOP_IPB)],
                            acc.at[idxd_v.at[j]], add=True)
        return carry

    lax.fori_loop(0, _HOP_CHUNKS, chunk, 0)
    plsc.subcore_barrier()
    pltpu.sync_copy(acc.at[pl.ds(r0, rows_per_tile)],
                    out_ref.at[c, pl.ds(r0, rows_per_tile)])


def _hop(x16, src2, dst2, zeros16):
    mesh = plsc.VectorSubcoreMesh(core_axis_name="c", subcore_axis_name="s")
    f = pl.kernel(
        _hop_body,
        out_type=jax.ShapeDtypeStruct((2, _NP, 16), jnp.float32),
        mesh=mesh,
        scratch_types=[
            pltpu.VMEM((_HOP_K, _HOP_IPB), jnp.int32),
            pltpu.VMEM((_HOP_K, _HOP_IPB), jnp.int32),
            pltpu.VMEM((_HOP_K * _HOP_IPB, 16), jnp.float32),
            pltpu.VMEM_SHARED((_NP, 16), jnp.float32),
            pltpu.SemaphoreType.DMA,
        ],
        compiler_params=pltpu.CompilerParams(use_tc_tiling_on_sc=False, needs_layout_passes=False),
    )
    return f(x16, src2, dst2, zeros16)


def _merge_body(a_ref, b_ref, o_ref):
    o_ref[...] = a_ref[0] + b_ref[0]


def _merge(p):
    blk = 6256
    return pl.pallas_call(
        _merge_body,
        grid=(_NP // blk,),
        in_specs=[
            pl.BlockSpec((1, blk, 16), lambda i: (0, i, 0)),
            pl.BlockSpec((1, blk, 16), lambda i: (1, i, 0)),
        ],
        out_specs=pl.BlockSpec((blk, 16), lambda i: (i, 0)),
        out_shape=jax.ShapeDtypeStruct((_NP, 16), jnp.float32),
    )(p, p)


_EP = 1605632
_AT_C = 256
_AT_CHUNKS = (_EP // 32) // _AT_C
_SM_C = 512
_SM_CHUNKS = (_EP // 32) // _SM_C
_NPT = _NP // 16
_ISQ = 0.22360679774997896


def _iota16():
    return lax.iota(jnp.int32, 16)


def _pass1_body(kv_ref, q_ref, as_ref, ad_ref, ea_ref,
                src_ref, dst_ref, srcn_ref, dstn_ref,
                msga_ref, msgb_ref, alpha_ref,
                si_v, di_v, ni_v, mi_v, kv_v, q_v, as_v, ad_v, ea_v,
                msga_v, msgb_v, al_v, sem):
    c = lax.axis_index("c")
    s = lax.axis_index("s")
    wid = s * 2 + c
    edge0 = wid * (_AT_CHUNKS * _AT_C)
    irow0 = edge0 // 128

    def chunk(ch, carry):
        base = edge0 + ch * _AT_C
        rb = irow0 + ch * (_AT_C // 128)
        pltpu.sync_copy(src_ref.at[pl.ds(rb, 2)], si_v)
        pltpu.sync_copy(dst_ref.at[pl.ds(rb, 2)], di_v)
        pltpu.sync_copy(srcn_ref.at[pl.ds(rb, 2)], ni_v)
        pltpu.sync_copy(dstn_ref.at[pl.ds(rb, 2)], mi_v)
        cps = []
        for j in range(2):
            sl = pl.ds(j * 128, 128)
            cps.append(pltpu.async_copy(kv_ref.at[si_v.at[j]], kv_v.at[sl], sem))
            cps.append(pltpu.async_copy(q_ref.at[di_v.at[j]], q_v.at[sl], sem))
            cps.append(pltpu.async_copy(as_ref.at[ni_v.at[j]], as_v.at[sl], sem))
            cps.append(pltpu.async_copy(ad_ref.at[mi_v.at[j]], ad_v.at[sl], sem))
        pltpu.sync_copy(ea_ref.at[pl.ds(base, _AT_C)], ea_v)
        for cp in cps:
            cp.wait()

        def block(g, carry2):
            i16 = _iota16() + g * 16
            acc = jnp.zeros((16,), jnp.float32)
            for d in range(EMB):
                cd = jnp.full((16,), d, jnp.int32)
                e_d = (plsc.load_gather(ea_v, [i16, cd])
                       + plsc.load_gather(as_v, [i16, cd])
                       + plsc.load_gather(ad_v, [i16, cd]))
                k_d = plsc.load_gather(kv_v, [i16, cd])
                v_d = plsc.load_gather(kv_v, [i16, jnp.full((16,), EMB + d, jnp.int32)])
                q_d = plsc.load_gather(q_v, [i16, cd])
                acc = acc + q_d * (k_d + e_d)
                if d < 16:
                    plsc.store_scatter(msga_v, [i16, cd], v_d + e_d)
                else:
                    cb = jnp.full((16,), d - 16, jnp.int32)
                    plsc.store_scatter(msgb_v, [i16, cb], v_d + e_d)
            z16 = jnp.zeros((16,), jnp.float32)
            for d in range(4, 8):
                plsc.store_scatter(msgb_v, [i16, jnp.full((16,), d, jnp.int32)], z16)
            al_v[pl.ds(g * 16, 16)] = acc * _ISQ
            return carry2

        lax.fori_loop(0, _AT_C // 16, block, 0)
        pltpu.sync_copy(msga_v, msga_ref.at[pl.ds(base, _AT_C)])
        pltpu.sync_copy(msgb_v, msgb_ref.at[pl.ds(base, _AT_C)])
        pltpu.sync_copy(al_v, alpha_ref.at[pl.ds(base, _AT_C)])
        return carry

    lax.fori_loop(0, _AT_CHUNKS, chunk, 0)


def _pass1(kv48, q32, as32, ad32, ea20, src2, dst2, srcn2, dstn2):
    mesh = plsc.VectorSubcoreMesh(core_axis_name="c", subcore_axis_name="s")
    f = pl.kernel(
        _pass1_body,
        out_type=(jax.ShapeDtypeStruct((_EP, 16), jnp.float32),
                  jax.ShapeDtypeStruct((_EP, 8), jnp.float32),
                  jax.ShapeDtypeStruct((_EP,), jnp.float32)),
        mesh=mesh,
        scratch_types=[
            pltpu.VMEM((2, 128), jnp.int32),
            pltpu.VMEM((2, 128), jnp.int32),
            pltpu.VMEM((2, 128), jnp.int32),
            pltpu.VMEM((2, 128), jnp.int32),
            pltpu.VMEM((_AT_C, 48), jnp.float32),
            pltpu.VMEM((_AT_C, 32), jnp.float32),
            pltpu.VMEM((_AT_C, 32), jnp.float32),
            pltpu.VMEM((_AT_C, 32), jnp.float32),
            pltpu.VMEM((_AT_C, EMB), jnp.float32),
            pltpu.VMEM((_AT_C, 16), jnp.float32),
            pltpu.VMEM((_AT_C, 8), jnp.float32),
            pltpu.VMEM((_AT_C,), jnp.float32),
            pltpu.SemaphoreType.DMA,
        ],
        compiler_params=pltpu.CompilerParams(use_tc_tiling_on_sc=False, needs_layout_passes=False),
    )
    return f(kv48, q32, as32, ad32, ea20, src2, dst2, srcn2, dstn2)


def _seg_max_body(alpha_ref, dst_ref, amax_ref,
                  maxtab, a_v, d_v, sem):
    c = lax.axis_index("c")
    s = lax.axis_index("s")
    wid = s * 2 + c

    def initb(i, carry):
        maxtab[pl.ds(i * 16, 16)] = jnp.full((16,), -3e38, jnp.float32)
        return carry

    lax.fori_loop(0, _NP // 16, initb, 0)

    edge0 = wid * (_SM_CHUNKS * _SM_C)
    irow0 = edge0 // 128

    def chunk(ch, carry):
        base = edge0 + ch * _SM_C
        rb = irow0 + ch * (_SM_C // 128)
        pltpu.sync_copy(alpha_ref.at[pl.ds(base, _SM_C)], a_v)
        pltpu.sync_copy(dst_ref.at[pl.ds(rb, _SM_C // 128)], d_v)

        def block(g, carry2):
            a16 = a_v[pl.ds(g * 16, 16)]
            j = g // 8
            o = (g % 8) * 16
            d16 = d_v[j, pl.ds(o, 16)]
            m0 = jnp.maximum(plsc.load_gather(maxtab, [d16]), a16)

            def cond(st):
                return st[1]

            def bodyw(st):
                m, _ = st
                plsc.store_scatter(maxtab, [d16], m)
                cur = plsc.load_gather(maxtab, [d16])
                return (jnp.maximum(cur, m), jnp.any(cur < m))

            lax.while_loop(cond, bodyw, (m0, True))
            return carry2

        lax.fori_loop(0, _SM_C // 16, block, 0)
        return carry

    lax.fori_loop(0, _SM_CHUNKS, chunk, 0)
    pltpu.sync_copy(maxtab, amax_ref.at[wid])


def _seg_max(alpha, dst2):
    mesh = plsc.VectorSubcoreMesh(core_axis_name="c", subcore_axis_name="s")
    f = pl.kernel(
        _seg_max_body,
        out_type=jax.ShapeDtypeStruct((32, _NP), jnp.float32),
        mesh=mesh,
        scratch_types=[
            pltpu.VMEM((_NP,), jnp.float32),
            pltpu.VMEM((_SM_C,), jnp.float32),
            pltpu.VMEM((_SM_C // 128, 128), jnp.int32),
            pltpu.SemaphoreType.DMA,
        ],
        compiler_params=pltpu.CompilerParams(use_tc_tiling_on_sc=False, needs_layout_passes=False),
    )
    return f(alpha, dst2)


def _amax_merge_body(p_ref, o_ref):
    o_ref[...] = jnp.max(p_ref[...], axis=0)


def _amax_merge(amax32):
    return pl.pallas_call(
        _amax_merge_body,
        out_shape=jax.ShapeDtypeStruct((_NP,), jnp.float32),
    )(amax32)


def _exp_den_body(alpha_ref, dst_ref, amax_ref, zeros_ref, al_ref, den_ref,
                  maxtab, a_v, d_v, al_v, den_sh, sem):
    c = lax.axis_index("c")
    s = lax.axis_index("s")
    wid = s * 2 + c
    r0 = s * _NPT
    pltpu.sync_copy(zeros_ref.at[pl.ds(r0, _NPT)], den_sh.at[pl.ds(r0, _NPT)])
    pltpu.sync_copy(amax_ref, maxtab)
    plsc.subcore_barrier()

    edge0 = wid * (_SM_CHUNKS * _SM_C)
    irow0 = edge0 // 128

    def chunk(ch, carry):
        base = edge0 + ch * _SM_C
        rb = irow0 + ch * (_SM_C // 128)
        pltpu.sync_copy(alpha_ref.at[pl.ds(base, _SM_C)], a_v)
        pltpu.sync_copy(dst_ref.at[pl.ds(rb, _SM_C // 128)], d_v)

        def block(g, carry2):
            a16 = a_v[pl.ds(g * 16, 16)]
            j = g // 8
            o = (g % 8) * 16
            d16 = d_v[j, pl.ds(o, 16)]
            mx16 = plsc.load_gather(maxtab, [d16])
            al_v[pl.ds(g * 16, 16)] = jnp.exp(a16 - mx16)
            return carry2

        lax.fori_loop(0, _SM_C // 16, block, 0)
        pltpu.sync_copy(al_v, al_ref.at[pl.ds(base, _SM_C)])
        for j in range(_SM_C // 128):
            pltpu.sync_copy(al_v.at[pl.ds(j * 128, 128)],
                            den_sh.at[d_v.at[j]], add=True)
        return carry

    lax.fori_loop(0, _SM_CHUNKS, chunk, 0)
    plsc.subcore_barrier()
    pltpu.sync_copy(den_sh.at[pl.ds(r0, _NPT)], den_ref.at[c, pl.ds(r0, _NPT)])


def _exp_den(alpha, dst2, amax, zerosN):
    mesh = plsc.VectorSubcoreMesh(core_axis_name="c", subcore_axis_name="s")
    f = pl.kernel(
        _exp_den_body,
        out_type=(jax.ShapeDtypeStruct((_EP,), jnp.float32),
                  jax.ShapeDtypeStruct((2, _NP), jnp.float32)),
        mesh=mesh,
        scratch_types=[
            pltpu.VMEM((_NP,), jnp.float32),
            pltpu.VMEM((_SM_C,), jnp.float32),
            pltpu.VMEM((_SM_C // 128, 128), jnp.int32),
            pltpu.VMEM((_SM_C,), jnp.float32),
            pltpu.VMEM_SHARED((_NP,), jnp.float32),
            pltpu.SemaphoreType.DMA,
        ],
        compiler_params=pltpu.CompilerParams(use_tc_tiling_on_sc=False, needs_layout_passes=False),
    )
    return f(alpha, dst2, amax, zerosN)


def _scatter_msg_body(w, msg_ref, al_ref, dst_ref, zeros_ref, out_ref,
                      m_v, al_v, d_v, w_v, acc, sem):
    c = lax.axis_index("c")
    s = lax.axis_index("s")
    wid = s * 2 + c
    r0 = s * _NPT
    pltpu.sync_copy(zeros_ref.at[pl.ds(r0, _NPT)], acc.at[pl.ds(r0, _NPT)])
    plsc.subcore_barrier()

    edge0 = wid * (_AT_CHUNKS * _AT_C)
    irow0 = edge0 // 128

    def chunk(ch, carry):
        base = edge0 + ch * _AT_C
        rb = irow0 + ch * (_AT_C // 128)
        pltpu.sync_copy(msg_ref.at[pl.ds(base, _AT_C)], m_v)
        pltpu.sync_copy(al_ref.at[pl.ds(base, _AT_C)], al_v)
        pltpu.sync_copy(dst_ref.at[pl.ds(rb, _AT_C // 128)], d_v)

        def block(g, carry2):
            i16 = _iota16() + g * 16
            w16 = al_v[pl.ds(g * 16, 16)]
            for d in range(w):
                cd = jnp.full((16,), d, jnp.int32)
                plsc.store_scatter(w_v, [i16, cd],
                                   plsc.load_gather(m_v, [i16, cd]) * w16)
            return carry2

        lax.fori_loop(0, _AT_C // 16, block, 0)
        for j in range(_AT_C // 128):
            pltpu.sync_copy(w_v.at[pl.ds(j * 128, 128)],
                            acc.at[d_v.at[j]], add=True)
        return carry

    lax.fori_loop(0, _AT_CHUNKS, chunk, 0)
    plsc.subcore_barrier()
    pltpu.sync_copy(acc.at[pl.ds(r0, _NPT)], out_ref.at[c, pl.ds(r0, _NPT)])


def _scatter_msg(msg, al, dst2, zerosNW, w):
    mesh = plsc.VectorSubcoreMesh(core_axis_name="c", subcore_axis_name="s")
    f = pl.kernel(
        functools.partial(_scatter_msg_body, w),
        out_type=jax.ShapeDtypeStruct((2, _NP, w), jnp.float32),
        mesh=mesh,
        scratch_types=[
            pltpu.VMEM((_AT_C, w), jnp.float32),
            pltpu.VMEM((_AT_C,), jnp.float32),
            pltpu.VMEM((_AT_C // 128, 128), jnp.int32),
            pltpu.VMEM((_AT_C, w), jnp.float32),
            pltpu.VMEM_SHARED((_NP, w), jnp.float32),
            pltpu.SemaphoreType.DMA,
        ],
        compiler_params=pltpu.CompilerParams(use_tc_tiling_on_sc=False, needs_layout_passes=False),
    )
    return f(msg, al, dst2, zerosNW)


def _final_body(pa_ref, pb_ref, den_ref, skip_ref, o_ref):
    a = jnp.concatenate(
        [pa_ref[0] + pa_ref[1], (pb_ref[0] + pb_ref[1])[:, :4]], axis=-1)
    dn = den_ref[:, 0] + den_ref[:, 1]
    safe = jnp.where(dn != 0.0, dn, 1.0)[:, None]
    o_ref[...] = jnp.where(dn[:, None] != 0.0, a / safe, 0.0) + skip_ref[...]


def _final(out_pa, out_pb, den_t, skip):
    blk = 2000
    return pl.pallas_call(
        _final_body,
        grid=(N // blk,),
        in_specs=[
            pl.BlockSpec((2, blk, 16), lambda i: (0, i, 0)),
            pl.BlockSpec((2, blk, 8), lambda i: (0, i, 0)),
            pl.BlockSpec((blk, 2), lambda i: (i, 0)),
            pl.BlockSpec((blk, EMB), lambda i: (i, 0)),
        ],
        out_specs=pl.BlockSpec((blk, EMB), lambda i: (i, 0)),
        out_shape=jax.ShapeDtypeStruct((N, EMB), jnp.float32),
    )(out_pa, out_pb, den_t, skip)


_NP_BLK = 3128


def _node_prep_body(x3f_ref, enc_ref, z_ref, W3T_ref, W1Tn_ref, W4T_ref,
                    W2Tn_ref, WqT_ref, WkT_ref, WvT_ref, WsT_ref, b_ref,
                    kv_ref, q_ref, as_ref, ad_ref, skip_ref):
    x3 = x3f_ref[:, :NODE_DIM]
    enc = enc_ref[...]
    z = z_ref[...]
    dot = functools.partial(jnp.dot, preferred_element_type=jnp.float32)
    As = dot(x3, W3T_ref[...]) + dot(enc, W1Tn_ref[...])
    Ad = dot(x3, W4T_ref[...]) + dot(enc, W2Tn_ref[...])
    q = dot(z, WqT_ref[...]) + b_ref[0:1, :]
    k = dot(z, WkT_ref[...]) + b_ref[1:2, :]
    v = dot(z, WvT_ref[...]) + b_ref[2:3, :]
    skip = dot(z, WsT_ref[...]) + b_ref[3:4, :]
    zpad = jnp.zeros((_NP_BLK, 12), jnp.float32)
    kv_ref[...] = jnp.concatenate([k, v, zpad[:, :8]], axis=-1)
    q_ref[...] = jnp.concatenate([q, zpad], axis=-1)
    as_ref[...] = jnp.concatenate([As, zpad], axis=-1)
    ad_ref[...] = jnp.concatenate([Ad, zpad], axis=-1)
    skip_ref[...] = skip


def _node_prep(x3f, encp, zp, W3T, W1Tn, W4T, W2Tn, WqT, WkT, WvT, WsT, b4):
    grid = (_NP // _NP_BLK,)
    w20 = lambda i: (0, 0)
    out = pl.pallas_call(
        _node_prep_body,
        grid=grid,
        in_specs=[
            pl.BlockSpec((_NP_BLK, 16), lambda i: (i, 0)),
            pl.BlockSpec((_NP_BLK, TIME), lambda i: (i, 0)),
            pl.BlockSpec((_NP_BLK, EMB), lambda i: (i, 0)),
            pl.BlockSpec((NODE_DIM, EMB), w20),
            pl.BlockSpec((TIME, EMB), w20),
            pl.BlockSpec((NODE_DIM, EMB), w20),
            pl.BlockSpec((TIME, EMB), w20),
            pl.BlockSpec((EMB, EMB), w20),
            pl.BlockSpec((EMB, EMB), w20),
            pl.BlockSpec((EMB, EMB), w20),
            pl.BlockSpec((EMB, EMB), w20),
            pl.BlockSpec((4, EMB), w20),
        ],
        out_specs=[
            pl.BlockSpec((_NP_BLK, 48), lambda i: (i, 0)),
            pl.BlockSpec((_NP_BLK, 32), lambda i: (i, 0)),
            pl.BlockSpec((_NP_BLK, 32), lambda i: (i, 0)),
            pl.BlockSpec((_NP_BLK, 32), lambda i: (i, 0)),
            pl.BlockSpec((_NP_BLK, EMB), lambda i: (i, 0)),
        ],
        out_shape=[
            jax.ShapeDtypeStruct((_NP, 48), jnp.float32),
            jax.ShapeDtypeStruct((_NP, 32), jnp.float32),
            jax.ShapeDtypeStruct((_NP, 32), jnp.float32),
            jax.ShapeDtypeStruct((_NP, 32), jnp.float32),
            jax.ShapeDtypeStruct((_NP, EMB), jnp.float32),
        ],
    )(x3f, encp, zp, W3T, W1Tn, W4T, W2Tn, WqT, WkT, WvT, WsT, b4)
    return out


def _edge_prep_body(ea_ref, t_ref, W0T_ref, s12_ref, out_ref):
    ea = ea_ref[...]
    t = t_ref[...]
    out_ref[...] = (
        jnp.dot(ea, W0T_ref[...], preferred_element_type=jnp.float32)
        + t * s12_ref[...]
    )


def _edge_prep(edge_attr_p, t_p, W0T, s12):
    blk = 8192
    grid = (_EP // blk,)
    out = pl.pallas_call(
        _edge_prep_body,
        grid=grid,
        in_specs=[
            pl.BlockSpec((blk, EDGE_DIM), lambda i: (i, 0)),
            pl.BlockSpec((blk, 1), lambda i: (i, 0)),
            pl.BlockSpec((EDGE_DIM, EMB), lambda i: (0, 0)),
            pl.BlockSpec((1, EMB), lambda i: (0, 0)),
        ],
        out_specs=pl.BlockSpec((blk, EMB), lambda i: (i, 0)),
        out_shape=jax.ShapeDtypeStruct((_EP, EMB), jnp.float32),
    )(edge_attr_p, t_p, W0T, s12)
    return out


def kernel(x, n_id, src_n_id, dst_n_id, edge_index, edge_attr, t,
           his_edge_index, enc_t_table, z,
           Wq, bq, Wk, bk, Wv, bv, We, Ws, bs):
    pad_idx = jnp.full((_MP - _M,), N, dtype=jnp.int32)
    src2 = jnp.concatenate(
        [edge_index[0], his_edge_index[0], pad_idx]).reshape(_MP // 128, 128)
    dst2 = jnp.concatenate(
        [edge_index[1], his_edge_index[1], pad_idx]).reshape(_MP // 128, 128)
    zeros16 = jnp.zeros((_NP, 16), jnp.float32)
    x16 = jnp.pad(x, ((0, _NP - N), (0, 16 - NODE_DIM)))
    for _ in range(3):
        x16 = _merge(_hop(x16, src2, dst2, zeros16))

    W0 = We[:, :EDGE_DIM]
    W1 = We[:, EDGE_DIM:EDGE_DIM + TIME]
    W2 = We[:, EDGE_DIM + TIME:EDGE_DIM + 2 * TIME]
    W3 = We[:, EDGE_DIM + 2 * TIME:EDGE_DIM + 2 * TIME + NODE_DIM]
    W4 = We[:, EDGE_DIM + 2 * TIME + NODE_DIM:]
    s12 = jnp.sum(W1 + W2, axis=1)[None, :]
    b4 = jnp.stack([bq, bk, bv, bs], axis=0)

    encp = jnp.pad(enc_t_table, ((0, _NP - N), (0, 0)))
    zp = jnp.pad(z, ((0, _NP - N), (0, 0)))
    kv48, q32, as32, ad32, skip = _node_prep(
        x16, encp, zp, W3.T, -W1.T, W4.T, -W2.T, Wq.T, Wk.T, Wv.T, Ws.T, b4)

    epad = _EP - E
    ea_p = jnp.pad(edge_attr, ((0, epad), (0, 0)))
    t_p = jnp.pad(t, (0, epad))[:, None]
    EA20 = _edge_prep(ea_p, t_p, W0.T, s12)

    e_pad_idx = jnp.full((epad,), N, dtype=jnp.int32)
    srcA = jnp.concatenate([edge_index[0], e_pad_idx]).reshape(_EP // 128, 128)
    dstA = jnp.concatenate([edge_index[1], e_pad_idx]).reshape(_EP // 128, 128)
    srcnA = jnp.concatenate([src_n_id, e_pad_idx]).reshape(_EP // 128, 128)
    dstnA = jnp.concatenate([dst_n_id, e_pad_idx]).reshape(_EP // 128, 128)

    msgA, msgB, alpha = _pass1(kv48, q32, as32, ad32, EA20,
                               srcA, dstA, srcnA, dstnA)
    dst = edge_index[1]
    alpha_x = alpha[:E]
    amax_x = jax.ops.segment_max(alpha_x, dst, num_segments=N)
    amax_x = jnp.where(jnp.isfinite(amax_x), amax_x, 0.0)
    al_x = jnp.exp(alpha_x - amax_x[dst])
    den_x = jax.ops.segment_sum(al_x, dst, num_segments=N)
    msg_x = jnp.concatenate([msgA[:E], msgB[:E, :4]], axis=1) * al_x[:, None]
    out_pre = jax.ops.segment_sum(msg_x, dst, num_segments=N)
    return (jnp.where(den_x[:, None] != 0.0, out_pre / den_x[:, None], 0.0)
            + skip[:N])
    amax = _amax_merge(_seg_max(alpha, dstA))
    zerosN = jnp.zeros((_NP,), jnp.float32)
    al, den_p = _exp_den(alpha, dstA, amax, zerosN)
    out_pa = _scatter_msg(msgA, al, dstA, zeros16, 16)
    zerosN8 = jnp.zeros((_NP, 8), jnp.float32)
    out_pb = _scatter_msg(msgB, al, dstA, zerosN8, 8)
    return _final(out_pa, out_pb, den_p.T, skip[:N])


# speedup vs baseline: 26.1315x; 2.6452x over previous
import functools

import jax
import jax.numpy as jnp
from jax import lax
from jax.experimental import pallas as pl
from jax.experimental.pallas import tpu as pltpu
from jax.experimental.pallas import tpu_sc as plsc

N = 100000
NODE_DIM = 10
EDGE_DIM = 16
EMB = 20
TIME = 10
E = 1600000

_NODE_BLK = 2000
_EDGE_BLK = 8000

_NP = 100096
_M = 2 * E
_MP = 3276800
_HOP_IPB = 128
_HOP_K = 4
_HOP_CHUNKS = (_MP // 32) // (_HOP_K * _HOP_IPB)


def _hop_body(x_ref, src_ref, dst_ref, zeros_ref, out_ref,
              idxs_v, idxd_v, rows_v, acc, sem):
    c = lax.axis_index("c")
    s = lax.axis_index("s")
    wid = s * 2 + c
    rows_per_tile = _NP // 16
    r0 = s * rows_per_tile
    pltpu.sync_copy(zeros_ref.at[pl.ds(r0, rows_per_tile)],
                    acc.at[pl.ds(r0, rows_per_tile)])
    plsc.subcore_barrier()

    idx_row0 = wid * (_HOP_CHUNKS * _HOP_K)

    def chunk(ch, carry):
        rb = idx_row0 + ch * _HOP_K
        pltpu.sync_copy(src_ref.at[pl.ds(rb, _HOP_K)], idxs_v)
        cps = [pltpu.async_copy(x_ref.at[idxs_v.at[j]],
                                rows_v.at[pl.ds(j * _HOP_IPB, _HOP_IPB)], sem)
               for j in range(_HOP_K)]
        for cp in cps:
            cp.wait()
        pltpu.sync_copy(dst_ref.at[pl.ds(rb, _HOP_K)], idxd_v)
        for j in range(_HOP_K):
            pltpu.sync_copy(rows_v.at[pl.ds(j * _HOP_IPB, _HOP_IPB)],
                            acc.at[idxd_v.at[j]], add=True)
        return carry

    lax.fori_loop(0, _HOP_CHUNKS, chunk, 0)
    plsc.subcore_barrier()
    pltpu.sync_copy(acc.at[pl.ds(r0, rows_per_tile)],
                    out_ref.at[c, pl.ds(r0, rows_per_tile)])


def _hop(x16, src2, dst2, zeros16):
    mesh = plsc.VectorSubcoreMesh(core_axis_name="c", subcore_axis_name="s")
    f = pl.kernel(
        _hop_body,
        out_type=jax.ShapeDtypeStruct((2, _NP, 16), jnp.float32),
        mesh=mesh,
        scratch_types=[
            pltpu.VMEM((_HOP_K, _HOP_IPB), jnp.int32),
            pltpu.VMEM((_HOP_K, _HOP_IPB), jnp.int32),
            pltpu.VMEM((_HOP_K * _HOP_IPB, 16), jnp.float32),
            pltpu.VMEM_SHARED((_NP, 16), jnp.float32),
            pltpu.SemaphoreType.DMA,
        ],
        compiler_params=pltpu.CompilerParams(use_tc_tiling_on_sc=False, needs_layout_passes=False),
    )
    return f(x16, src2, dst2, zeros16)


def _merge_body(a_ref, b_ref, o_ref):
    o_ref[...] = a_ref[0] + b_ref[0]


def _merge(p):
    blk = 6256
    return pl.pallas_call(
        _merge_body,
        grid=(_NP // blk,),
        in_specs=[
            pl.BlockSpec((1, blk, 16), lambda i: (0, i, 0)),
            pl.BlockSpec((1, blk, 16), lambda i: (1, i, 0)),
        ],
        out_specs=pl.BlockSpec((blk, 16), lambda i: (i, 0)),
        out_shape=jax.ShapeDtypeStruct((_NP, 16), jnp.float32),
    )(p, p)


_EP = 1605632
_AT_C = 256
_AT_CHUNKS = (_EP // 32) // _AT_C
_SM_C = 512
_SM_CHUNKS = (_EP // 32) // _SM_C
_NPT = _NP // 16
_ISQ = 0.22360679774997896


def _iota16():
    return lax.iota(jnp.int32, 16)


def _pass1_body(kv_ref, q_ref, as_ref, ad_ref, ea_ref,
                src_ref, dst_ref, srcn_ref, dstn_ref,
                msga_ref, msgb_ref, alpha_ref,
                si_v, di_v, ni_v, mi_v, kv_v, q_v, as_v, ad_v, ea_v,
                msga_v, msgb_v, al_v, sem):
    c = lax.axis_index("c")
    s = lax.axis_index("s")
    wid = s * 2 + c
    edge0 = wid * (_AT_CHUNKS * _AT_C)
    irow0 = edge0 // 128

    def chunk(ch, carry):
        base = edge0 + ch * _AT_C
        rb = irow0 + ch * (_AT_C // 128)
        pltpu.sync_copy(src_ref.at[pl.ds(rb, 2)], si_v)
        pltpu.sync_copy(dst_ref.at[pl.ds(rb, 2)], di_v)
        pltpu.sync_copy(srcn_ref.at[pl.ds(rb, 2)], ni_v)
        pltpu.sync_copy(dstn_ref.at[pl.ds(rb, 2)], mi_v)
        cps = []
        for j in range(2):
            sl = pl.ds(j * 128, 128)
            cps.append(pltpu.async_copy(kv_ref.at[si_v.at[j]], kv_v.at[sl], sem))
            cps.append(pltpu.async_copy(q_ref.at[di_v.at[j]], q_v.at[sl], sem))
            cps.append(pltpu.async_copy(as_ref.at[ni_v.at[j]], as_v.at[sl], sem))
            cps.append(pltpu.async_copy(ad_ref.at[mi_v.at[j]], ad_v.at[sl], sem))
        pltpu.sync_copy(ea_ref.at[pl.ds(base, _AT_C)], ea_v)
        for cp in cps:
            cp.wait()

        def block(g, carry2):
            i16 = _iota16() + g * 16
            acc = jnp.zeros((16,), jnp.float32)
            for d in range(EMB):
                cd = jnp.full((16,), d, jnp.int32)
                e_d = (plsc.load_gather(ea_v, [i16, cd])
                       + plsc.load_gather(as_v, [i16, cd])
                       + plsc.load_gather(ad_v, [i16, cd]))
                k_d = plsc.load_gather(kv_v, [i16, cd])
                v_d = plsc.load_gather(kv_v, [i16, jnp.full((16,), EMB + d, jnp.int32)])
                q_d = plsc.load_gather(q_v, [i16, cd])
                acc = acc + q_d * (k_d + e_d)
                if d < 16:
                    plsc.store_scatter(msga_v, [i16, cd], v_d + e_d)
                else:
                    cb = jnp.full((16,), d - 16, jnp.int32)
                    plsc.store_scatter(msgb_v, [i16, cb], v_d + e_d)
            z16 = jnp.zeros((16,), jnp.float32)
            for d in range(4, 8):
                plsc.store_scatter(msgb_v, [i16, jnp.full((16,), d, jnp.int32)], z16)
            al_v[pl.ds(g * 16, 16)] = acc * _ISQ
            return carry2

        lax.fori_loop(0, _AT_C // 16, block, 0)
        pltpu.sync_copy(msga_v, msga_ref.at[pl.ds(base, _AT_C)])
        pltpu.sync_copy(msgb_v, msgb_ref.at[pl.ds(base, _AT_C)])
        pltpu.sync_copy(al_v, alpha_ref.at[pl.ds(base, _AT_C)])
        return carry

    lax.fori_loop(0, _AT_CHUNKS, chunk, 0)


def _pass1(kv48, q32, as32, ad32, ea20, src2, dst2, srcn2, dstn2):
    mesh = plsc.VectorSubcoreMesh(core_axis_name="c", subcore_axis_name="s")
    f = pl.kernel(
        _pass1_body,
        out_type=(jax.ShapeDtypeStruct((_EP, 16), jnp.float32),
                  jax.ShapeDtypeStruct((_EP, 8), jnp.float32),
                  jax.ShapeDtypeStruct((_EP,), jnp.float32)),
        mesh=mesh,
        scratch_types=[
            pltpu.VMEM((2, 128), jnp.int32),
            pltpu.VMEM((2, 128), jnp.int32),
            pltpu.VMEM((2, 128), jnp.int32),
            pltpu.VMEM((2, 128), jnp.int32),
            pltpu.VMEM((_AT_C, 48), jnp.float32),
            pltpu.VMEM((_AT_C, 32), jnp.float32),
            pltpu.VMEM((_AT_C, 32), jnp.float32),
            pltpu.VMEM((_AT_C, 32), jnp.float32),
            pltpu.VMEM((_AT_C, EMB), jnp.float32),
            pltpu.VMEM((_AT_C, 16), jnp.float32),
            pltpu.VMEM((_AT_C, 8), jnp.float32),
            pltpu.VMEM((_AT_C,), jnp.float32),
            pltpu.SemaphoreType.DMA,
        ],
        compiler_params=pltpu.CompilerParams(use_tc_tiling_on_sc=False, needs_layout_passes=False),
    )
    return f(kv48, q32, as32, ad32, ea20, src2, dst2, srcn2, dstn2)


def _seg_max_body(alpha_ref, dst_ref, amax_ref,
                  maxtab, a_v, d_v, sem):
    c = lax.axis_index("c")
    s = lax.axis_index("s")
    wid = s * 2 + c

    def initb(i, carry):
        maxtab[pl.ds(i * 16, 16)] = jnp.full((16,), -3e38, jnp.float32)
        return carry

    lax.fori_loop(0, _NP // 16, initb, 0)

    edge0 = wid * (_SM_CHUNKS * _SM_C)
    irow0 = edge0 // 128

    def chunk(ch, carry):
        base = edge0 + ch * _SM_C
        rb = irow0 + ch * (_SM_C // 128)
        pltpu.sync_copy(alpha_ref.at[pl.ds(base, _SM_C)], a_v)
        pltpu.sync_copy(dst_ref.at[pl.ds(rb, _SM_C // 128)], d_v)

        def block(g, carry2):
            a16 = a_v[pl.ds(g * 16, 16)]
            j = g // 8
            o = (g % 8) * 16
            d16 = d_v[j, pl.ds(o, 16)]

            def rnd(r, need):
                plsc.store_scatter(maxtab, [d16], a16, mask=need)
                cur = plsc.load_gather(maxtab, [d16])
                return jnp.logical_and(need, cur < a16)

            need0 = plsc.load_gather(maxtab, [d16]) < a16
            lax.fori_loop(0, 16, rnd, need0)
            return carry2

        lax.fori_loop(0, _SM_C // 16, block, 0)
        return carry

    lax.fori_loop(0, _SM_CHUNKS, chunk, 0)
    pltpu.sync_copy(maxtab, amax_ref.at[wid])


def _seg_max(alpha, dst2):
    mesh = plsc.VectorSubcoreMesh(core_axis_name="c", subcore_axis_name="s")
    f = pl.kernel(
        _seg_max_body,
        out_type=jax.ShapeDtypeStruct((32, _NP), jnp.float32),
        mesh=mesh,
        scratch_types=[
            pltpu.VMEM((_NP,), jnp.float32),
            pltpu.VMEM((_SM_C,), jnp.float32),
            pltpu.VMEM((_SM_C // 128, 128), jnp.int32),
            pltpu.SemaphoreType.DMA,
        ],
        compiler_params=pltpu.CompilerParams(use_tc_tiling_on_sc=False, needs_layout_passes=False),
    )
    return f(alpha, dst2)


def _amax_merge_body(p_ref, o_ref):
    o_ref[...] = jnp.max(p_ref[...], axis=0)


def _amax_merge(amax32):
    return pl.pallas_call(
        _amax_merge_body,
        out_shape=jax.ShapeDtypeStruct((_NP,), jnp.float32),
    )(amax32)


def _exp_den_body(alpha_ref, dst_ref, amax_ref, zeros_ref, al_ref, den_ref,
                  maxtab, a_v, d_v, al_v, den_sh, sem):
    c = lax.axis_index("c")
    s = lax.axis_index("s")
    wid = s * 2 + c
    r0 = s * _NPT
    pltpu.sync_copy(zeros_ref.at[pl.ds(r0, _NPT)], den_sh.at[pl.ds(r0, _NPT)])
    pltpu.sync_copy(amax_ref, maxtab)
    plsc.subcore_barrier()

    edge0 = wid * (_SM_CHUNKS * _SM_C)
    irow0 = edge0 // 128

    def chunk(ch, carry):
        base = edge0 + ch * _SM_C
        rb = irow0 + ch * (_SM_C // 128)
        pltpu.sync_copy(alpha_ref.at[pl.ds(base, _SM_C)], a_v)
        pltpu.sync_copy(dst_ref.at[pl.ds(rb, _SM_C // 128)], d_v)

        def block(g, carry2):
            a16 = a_v[pl.ds(g * 16, 16)]
            j = g // 8
            o = (g % 8) * 16
            d16 = d_v[j, pl.ds(o, 16)]
            mx16 = plsc.load_gather(maxtab, [d16])
            al_v[pl.ds(g * 16, 16)] = jnp.exp(a16 - mx16)
            return carry2

        lax.fori_loop(0, _SM_C // 16, block, 0)
        pltpu.sync_copy(al_v, al_ref.at[pl.ds(base, _SM_C)])
        for j in range(_SM_C // 128):
            pltpu.sync_copy(al_v.at[pl.ds(j * 128, 128)],
                            den_sh.at[d_v.at[j]], add=True)
        return carry

    lax.fori_loop(0, _SM_CHUNKS, chunk, 0)
    plsc.subcore_barrier()
    pltpu.sync_copy(den_sh.at[pl.ds(r0, _NPT)], den_ref.at[c, pl.ds(r0, _NPT)])


def _exp_den(alpha, dst2, amax, zerosN):
    mesh = plsc.VectorSubcoreMesh(core_axis_name="c", subcore_axis_name="s")
    f = pl.kernel(
        _exp_den_body,
        out_type=(jax.ShapeDtypeStruct((_EP,), jnp.float32),
                  jax.ShapeDtypeStruct((2, _NP), jnp.float32)),
        mesh=mesh,
        scratch_types=[
            pltpu.VMEM((_NP,), jnp.float32),
            pltpu.VMEM((_SM_C,), jnp.float32),
            pltpu.VMEM((_SM_C // 128, 128), jnp.int32),
            pltpu.VMEM((_SM_C,), jnp.float32),
            pltpu.VMEM_SHARED((_NP,), jnp.float32),
            pltpu.SemaphoreType.DMA,
        ],
        compiler_params=pltpu.CompilerParams(use_tc_tiling_on_sc=False, needs_layout_passes=False),
    )
    return f(alpha, dst2, amax, zerosN)


def _scatter_msg_body(w, msg_ref, al_ref, dst_ref, zeros_ref, out_ref,
                      m_v, al_v, d_v, w_v, acc, sem):
    c = lax.axis_index("c")
    s = lax.axis_index("s")
    wid = s * 2 + c
    r0 = s * _NPT
    pltpu.sync_copy(zeros_ref.at[pl.ds(r0, _NPT)], acc.at[pl.ds(r0, _NPT)])
    plsc.subcore_barrier()

    edge0 = wid * (_AT_CHUNKS * _AT_C)
    irow0 = edge0 // 128

    def chunk(ch, carry):
        base = edge0 + ch * _AT_C
        rb = irow0 + ch * (_AT_C // 128)
        pltpu.sync_copy(msg_ref.at[pl.ds(base, _AT_C)], m_v)
        pltpu.sync_copy(al_ref.at[pl.ds(base, _AT_C)], al_v)
        pltpu.sync_copy(dst_ref.at[pl.ds(rb, _AT_C // 128)], d_v)

        def block(g, carry2):
            i16 = _iota16() + g * 16
            w16 = al_v[pl.ds(g * 16, 16)]
            for d in range(w):
                cd = jnp.full((16,), d, jnp.int32)
                plsc.store_scatter(w_v, [i16, cd],
                                   plsc.load_gather(m_v, [i16, cd]) * w16)
            return carry2

        lax.fori_loop(0, _AT_C // 16, block, 0)
        for j in range(_AT_C // 128):
            pltpu.sync_copy(w_v.at[pl.ds(j * 128, 128)],
                            acc.at[d_v.at[j]], add=True)
        return carry

    lax.fori_loop(0, _AT_CHUNKS, chunk, 0)
    plsc.subcore_barrier()
    pltpu.sync_copy(acc.at[pl.ds(r0, _NPT)], out_ref.at[c, pl.ds(r0, _NPT)])


def _scatter_msg(msg, al, dst2, zerosNW, w):
    mesh = plsc.VectorSubcoreMesh(core_axis_name="c", subcore_axis_name="s")
    f = pl.kernel(
        functools.partial(_scatter_msg_body, w),
        out_type=jax.ShapeDtypeStruct((2, _NP, w), jnp.float32),
        mesh=mesh,
        scratch_types=[
            pltpu.VMEM((_AT_C, w), jnp.float32),
            pltpu.VMEM((_AT_C,), jnp.float32),
            pltpu.VMEM((_AT_C // 128, 128), jnp.int32),
            pltpu.VMEM((_AT_C, w), jnp.float32),
            pltpu.VMEM_SHARED((_NP, w), jnp.float32),
            pltpu.SemaphoreType.DMA,
        ],
        compiler_params=pltpu.CompilerParams(use_tc_tiling_on_sc=False, needs_layout_passes=False),
    )
    return f(msg, al, dst2, zerosNW)


def _final_body(pa_ref, pb_ref, den_ref, skip_ref, o_ref):
    a = jnp.concatenate(
        [pa_ref[0] + pa_ref[1], (pb_ref[0] + pb_ref[1])[:, :4]], axis=-1)
    dn = den_ref[:, 0] + den_ref[:, 1]
    safe = jnp.where(dn != 0.0, dn, 1.0)[:, None]
    o_ref[...] = jnp.where(dn[:, None] != 0.0, a / safe, 0.0) + skip_ref[...]


def _final(out_pa, out_pb, den_t, skip):
    blk = 2000
    return pl.pallas_call(
        _final_body,
        grid=(N // blk,),
        in_specs=[
            pl.BlockSpec((2, blk, 16), lambda i: (0, i, 0)),
            pl.BlockSpec((2, blk, 8), lambda i: (0, i, 0)),
            pl.BlockSpec((blk, 2), lambda i: (i, 0)),
            pl.BlockSpec((blk, EMB), lambda i: (i, 0)),
        ],
        out_specs=pl.BlockSpec((blk, EMB), lambda i: (i, 0)),
        out_shape=jax.ShapeDtypeStruct((N, EMB), jnp.float32),
    )(out_pa, out_pb, den_t, skip)


_NP_BLK = 3128


def _node_prep_body(x3f_ref, enc_ref, z_ref, W3T_ref, W1Tn_ref, W4T_ref,
                    W2Tn_ref, WqT_ref, WkT_ref, WvT_ref, WsT_ref, b_ref,
                    kv_ref, q_ref, as_ref, ad_ref, skip_ref):
    x3 = x3f_ref[:, :NODE_DIM]
    enc = enc_ref[...]
    z = z_ref[...]
    dot = functools.partial(jnp.dot, preferred_element_type=jnp.float32)
    As = dot(x3, W3T_ref[...]) + dot(enc, W1Tn_ref[...])
    Ad = dot(x3, W4T_ref[...]) + dot(enc, W2Tn_ref[...])
    q = dot(z, WqT_ref[...]) + b_ref[0:1, :]
    k = dot(z, WkT_ref[...]) + b_ref[1:2, :]
    v = dot(z, WvT_ref[...]) + b_ref[2:3, :]
    skip = dot(z, WsT_ref[...]) + b_ref[3:4, :]
    zpad = jnp.zeros((_NP_BLK, 12), jnp.float32)
    kv_ref[...] = jnp.concatenate([k, v, zpad[:, :8]], axis=-1)
    q_ref[...] = jnp.concatenate([q, zpad], axis=-1)
    as_ref[...] = jnp.concatenate([As, zpad], axis=-1)
    ad_ref[...] = jnp.concatenate([Ad, zpad], axis=-1)
    skip_ref[...] = skip


def _node_prep(x3f, encp, zp, W3T, W1Tn, W4T, W2Tn, WqT, WkT, WvT, WsT, b4):
    grid = (_NP // _NP_BLK,)
    w20 = lambda i: (0, 0)
    out = pl.pallas_call(
        _node_prep_body,
        grid=grid,
        in_specs=[
            pl.BlockSpec((_NP_BLK, 16), lambda i: (i, 0)),
            pl.BlockSpec((_NP_BLK, TIME), lambda i: (i, 0)),
            pl.BlockSpec((_NP_BLK, EMB), lambda i: (i, 0)),
            pl.BlockSpec((NODE_DIM, EMB), w20),
            pl.BlockSpec((TIME, EMB), w20),
            pl.BlockSpec((NODE_DIM, EMB), w20),
            pl.BlockSpec((TIME, EMB), w20),
            pl.BlockSpec((EMB, EMB), w20),
            pl.BlockSpec((EMB, EMB), w20),
            pl.BlockSpec((EMB, EMB), w20),
            pl.BlockSpec((EMB, EMB), w20),
            pl.BlockSpec((4, EMB), w20),
        ],
        out_specs=[
            pl.BlockSpec((_NP_BLK, 48), lambda i: (i, 0)),
            pl.BlockSpec((_NP_BLK, 32), lambda i: (i, 0)),
            pl.BlockSpec((_NP_BLK, 32), lambda i: (i, 0)),
            pl.BlockSpec((_NP_BLK, 32), lambda i: (i, 0)),
            pl.BlockSpec((_NP_BLK, EMB), lambda i: (i, 0)),
        ],
        out_shape=[
            jax.ShapeDtypeStruct((_NP, 48), jnp.float32),
            jax.ShapeDtypeStruct((_NP, 32), jnp.float32),
            jax.ShapeDtypeStruct((_NP, 32), jnp.float32),
            jax.ShapeDtypeStruct((_NP, 32), jnp.float32),
            jax.ShapeDtypeStruct((_NP, EMB), jnp.float32),
        ],
    )(x3f, encp, zp, W3T, W1Tn, W4T, W2Tn, WqT, WkT, WvT, WsT, b4)
    return out


def _edge_prep_body(ea_ref, t_ref, W0T_ref, s12_ref, out_ref):
    ea = ea_ref[...]
    t = t_ref[...]
    out_ref[...] = (
        jnp.dot(ea, W0T_ref[...], preferred_element_type=jnp.float32)
        + t * s12_ref[...]
    )


def _edge_prep(edge_attr_p, t_p, W0T, s12):
    blk = 8192
    grid = (_EP // blk,)
    out = pl.pallas_call(
        _edge_prep_body,
        grid=grid,
        in_specs=[
            pl.BlockSpec((blk, EDGE_DIM), lambda i: (i, 0)),
            pl.BlockSpec((blk, 1), lambda i: (i, 0)),
            pl.BlockSpec((EDGE_DIM, EMB), lambda i: (0, 0)),
            pl.BlockSpec((1, EMB), lambda i: (0, 0)),
        ],
        out_specs=pl.BlockSpec((blk, EMB), lambda i: (i, 0)),
        out_shape=jax.ShapeDtypeStruct((_EP, EMB), jnp.float32),
    )(edge_attr_p, t_p, W0T, s12)
    return out


def kernel(x, n_id, src_n_id, dst_n_id, edge_index, edge_attr, t,
           his_edge_index, enc_t_table, z,
           Wq, bq, Wk, bk, Wv, bv, We, Ws, bs):
    pad_idx = jnp.full((_MP - _M,), N, dtype=jnp.int32)
    src2 = jnp.concatenate(
        [edge_index[0], his_edge_index[0], pad_idx]).reshape(_MP // 128, 128)
    dst2 = jnp.concatenate(
        [edge_index[1], his_edge_index[1], pad_idx]).reshape(_MP // 128, 128)
    zeros16 = jnp.zeros((_NP, 16), jnp.float32)
    x16 = jnp.pad(x, ((0, _NP - N), (0, 16 - NODE_DIM)))
    for _ in range(3):
        x16 = _merge(_hop(x16, src2, dst2, zeros16))

    W0 = We[:, :EDGE_DIM]
    W1 = We[:, EDGE_DIM:EDGE_DIM + TIME]
    W2 = We[:, EDGE_DIM + TIME:EDGE_DIM + 2 * TIME]
    W3 = We[:, EDGE_DIM + 2 * TIME:EDGE_DIM + 2 * TIME + NODE_DIM]
    W4 = We[:, EDGE_DIM + 2 * TIME + NODE_DIM:]
    s12 = jnp.sum(W1 + W2, axis=1)[None, :]
    b4 = jnp.stack([bq, bk, bv, bs], axis=0)

    encp = jnp.pad(enc_t_table, ((0, _NP - N), (0, 0)))
    zp = jnp.pad(z, ((0, _NP - N), (0, 0)))
    kv48, q32, as32, ad32, skip = _node_prep(
        x16, encp, zp, W3.T, -W1.T, W4.T, -W2.T, Wq.T, Wk.T, Wv.T, Ws.T, b4)

    epad = _EP - E
    ea_p = jnp.pad(edge_attr, ((0, epad), (0, 0)))
    t_p = jnp.pad(t, (0, epad))[:, None]
    EA20 = _edge_prep(ea_p, t_p, W0.T, s12)

    e_pad_idx = jnp.full((epad,), N, dtype=jnp.int32)
    srcA = jnp.concatenate([edge_index[0], e_pad_idx]).reshape(_EP // 128, 128)
    dstA = jnp.concatenate([edge_index[1], e_pad_idx]).reshape(_EP // 128, 128)
    srcnA = jnp.concatenate([src_n_id, e_pad_idx]).reshape(_EP // 128, 128)
    dstnA = jnp.concatenate([dst_n_id, e_pad_idx]).reshape(_EP // 128, 128)

    msgA, msgB, alpha = _pass1(kv48, q32, as32, ad32, EA20,
                               srcA, dstA, srcnA, dstnA)
    amax = _amax_merge(_seg_max(alpha, dstA))
    zerosN = jnp.zeros((_NP,), jnp.float32)
    al, den_p = _exp_den(alpha, dstA, amax, zerosN)
    out_pa = _scatter_msg(msgA, al, dstA, zeros16, 16)
    zerosN8 = jnp.zeros((_NP, 8), jnp.float32)
    out_pb = _scatter_msg(msgB, al, dstA, zerosN8, 8)
    return _final(out_pa, out_pb, den_p.T, skip[:N])
    amax = _amax_merge(_seg_max(alpha, dstA))
    zerosN = jnp.zeros((_NP,), jnp.float32)
    al, den_p = _exp_den(alpha, dstA, amax, zerosN)
    out_pa = _scatter_msg(msgA, al, dstA, zeros16, 16)
    zerosN8 = jnp.zeros((_NP, 8), jnp.float32)
    out_pb = _scatter_msg(msgB, al, dstA, zerosN8, 8)
    return _final(out_pa, out_pb, den_p.T, skip[:N])


# speedup vs baseline: 27.2404x; 1.0424x over previous
import functools

import jax
import jax.numpy as jnp
from jax import lax
from jax.experimental import pallas as pl
from jax.experimental.pallas import tpu as pltpu
from jax.experimental.pallas import tpu_sc as plsc

N = 100000
NODE_DIM = 10
EDGE_DIM = 16
EMB = 20
TIME = 10
E = 1600000

_NODE_BLK = 2000
_EDGE_BLK = 8000

_NP = 100096
_M = 2 * E
_MP = 3276800
_HOP_IPB = 128
_HOP_K = 4
_HOP_CHUNKS = (_MP // 32) // (_HOP_K * _HOP_IPB)


def _hop_body(x_ref, src_ref, dst_ref, zeros_ref, out_ref,
              idxs_v, idxd_v, rows_v, acc, sem):
    c = lax.axis_index("c")
    s = lax.axis_index("s")
    wid = s * 2 + c
    rows_per_tile = _NP // 16
    r0 = s * rows_per_tile
    pltpu.sync_copy(zeros_ref.at[pl.ds(r0, rows_per_tile)],
                    acc.at[pl.ds(r0, rows_per_tile)])
    plsc.subcore_barrier()

    idx_row0 = wid * (_HOP_CHUNKS * _HOP_K)

    def chunk(ch, carry):
        rb = idx_row0 + ch * _HOP_K
        pltpu.sync_copy(src_ref.at[pl.ds(rb, _HOP_K)], idxs_v)
        cps = [pltpu.async_copy(x_ref.at[idxs_v.at[j]],
                                rows_v.at[pl.ds(j * _HOP_IPB, _HOP_IPB)], sem)
               for j in range(_HOP_K)]
        for cp in cps:
            cp.wait()
        pltpu.sync_copy(dst_ref.at[pl.ds(rb, _HOP_K)], idxd_v)
        for j in range(_HOP_K):
            pltpu.sync_copy(rows_v.at[pl.ds(j * _HOP_IPB, _HOP_IPB)],
                            acc.at[idxd_v.at[j]], add=True)
        return carry

    lax.fori_loop(0, _HOP_CHUNKS, chunk, 0)
    plsc.subcore_barrier()
    pltpu.sync_copy(acc.at[pl.ds(r0, rows_per_tile)],
                    out_ref.at[c, pl.ds(r0, rows_per_tile)])


def _hop(x16, src2, dst2, zeros16):
    mesh = plsc.VectorSubcoreMesh(core_axis_name="c", subcore_axis_name="s")
    f = pl.kernel(
        _hop_body,
        out_type=jax.ShapeDtypeStruct((2, _NP, 16), jnp.float32),
        mesh=mesh,
        scratch_types=[
            pltpu.VMEM((_HOP_K, _HOP_IPB), jnp.int32),
            pltpu.VMEM((_HOP_K, _HOP_IPB), jnp.int32),
            pltpu.VMEM((_HOP_K * _HOP_IPB, 16), jnp.float32),
            pltpu.VMEM_SHARED((_NP, 16), jnp.float32),
            pltpu.SemaphoreType.DMA,
        ],
        compiler_params=pltpu.CompilerParams(use_tc_tiling_on_sc=False, needs_layout_passes=False),
    )
    return f(x16, src2, dst2, zeros16)


def _merge_body(a_ref, b_ref, o_ref):
    o_ref[...] = a_ref[0] + b_ref[0]


def _merge(p):
    blk = 6256
    return pl.pallas_call(
        _merge_body,
        grid=(_NP // blk,),
        in_specs=[
            pl.BlockSpec((1, blk, 16), lambda i: (0, i, 0)),
            pl.BlockSpec((1, blk, 16), lambda i: (1, i, 0)),
        ],
        out_specs=pl.BlockSpec((blk, 16), lambda i: (i, 0)),
        out_shape=jax.ShapeDtypeStruct((_NP, 16), jnp.float32),
    )(p, p)


_EP = 1605632
_AT_C = 512
_AT_CHUNKS = (_EP // 32) // _AT_C
_SM_C = 512
_SM_CHUNKS = (_EP // 32) // _SM_C
_NPT = _NP // 16
_ISQ = 0.22360679774997896


def _iota16():
    return lax.iota(jnp.int32, 16)


def _pass1_body(kv_ref, q_ref, as_ref, ad_ref, ea_ref,
                src_ref, dst_ref, srcn_ref, dstn_ref,
                msga_ref, msgb_ref, alpha_ref,
                si_v, di_v, ni_v, mi_v, kv_v, q_v, as_v, ad_v, ea_v,
                msga_v, msgb_v, al_v, sem):
    c = lax.axis_index("c")
    s = lax.axis_index("s")
    wid = s * 2 + c
    edge0 = wid * (_AT_CHUNKS * _AT_C)
    irow0 = edge0 // 128

    def chunk(ch, carry):
        base = edge0 + ch * _AT_C
        rb = irow0 + ch * (_AT_C // 128)
        nj = _AT_C // 128
        pltpu.sync_copy(src_ref.at[pl.ds(rb, nj)], si_v)
        pltpu.sync_copy(dst_ref.at[pl.ds(rb, nj)], di_v)
        pltpu.sync_copy(srcn_ref.at[pl.ds(rb, nj)], ni_v)
        pltpu.sync_copy(dstn_ref.at[pl.ds(rb, nj)], mi_v)
        cps = []
        for j in range(nj):
            sl = pl.ds(j * 128, 128)
            cps.append(pltpu.async_copy(kv_ref.at[si_v.at[j]], kv_v.at[sl], sem))
            cps.append(pltpu.async_copy(q_ref.at[di_v.at[j]], q_v.at[sl], sem))
            cps.append(pltpu.async_copy(as_ref.at[ni_v.at[j]], as_v.at[sl], sem))
            cps.append(pltpu.async_copy(ad_ref.at[mi_v.at[j]], ad_v.at[sl], sem))
        pltpu.sync_copy(ea_ref.at[pl.ds(base, _AT_C)], ea_v)
        for cp in cps:
            cp.wait()

        def block(g, carry2):
            i16 = _iota16() + g * 16
            acc = jnp.zeros((16,), jnp.float32)
            for d in range(EMB):
                cd = jnp.full((16,), d, jnp.int32)
                e_d = (plsc.load_gather(ea_v, [i16, cd])
                       + plsc.load_gather(as_v, [i16, cd])
                       + plsc.load_gather(ad_v, [i16, cd]))
                k_d = plsc.load_gather(kv_v, [i16, cd])
                v_d = plsc.load_gather(kv_v, [i16, jnp.full((16,), EMB + d, jnp.int32)])
                q_d = plsc.load_gather(q_v, [i16, cd])
                acc = acc + q_d * (k_d + e_d)
                if d < 16:
                    plsc.store_scatter(msga_v, [i16, cd], v_d + e_d)
                else:
                    cb = jnp.full((16,), d - 16, jnp.int32)
                    plsc.store_scatter(msgb_v, [i16, cb], v_d + e_d)
            z16 = jnp.zeros((16,), jnp.float32)
            for d in range(4, 8):
                plsc.store_scatter(msgb_v, [i16, jnp.full((16,), d, jnp.int32)], z16)
            al_v[pl.ds(g * 16, 16)] = acc * _ISQ
            return carry2

        lax.fori_loop(0, _AT_C // 16, block, 0)
        pltpu.sync_copy(msga_v, msga_ref.at[pl.ds(base, _AT_C)])
        pltpu.sync_copy(msgb_v, msgb_ref.at[pl.ds(base, _AT_C)])
        pltpu.sync_copy(al_v, alpha_ref.at[pl.ds(base, _AT_C)])
        return carry

    lax.fori_loop(0, _AT_CHUNKS, chunk, 0)


def _pass1(kv48, q32, as32, ad32, ea20, src2, dst2, srcn2, dstn2):
    mesh = plsc.VectorSubcoreMesh(core_axis_name="c", subcore_axis_name="s")
    f = pl.kernel(
        _pass1_body,
        out_type=(jax.ShapeDtypeStruct((_EP, 16), jnp.float32),
                  jax.ShapeDtypeStruct((_EP, 8), jnp.float32),
                  jax.ShapeDtypeStruct((_EP,), jnp.float32)),
        mesh=mesh,
        scratch_types=[
            pltpu.VMEM((_AT_C // 128, 128), jnp.int32),
            pltpu.VMEM((_AT_C // 128, 128), jnp.int32),
            pltpu.VMEM((_AT_C // 128, 128), jnp.int32),
            pltpu.VMEM((_AT_C // 128, 128), jnp.int32),
            pltpu.VMEM((_AT_C, 48), jnp.float32),
            pltpu.VMEM((_AT_C, 32), jnp.float32),
            pltpu.VMEM((_AT_C, 32), jnp.float32),
            pltpu.VMEM((_AT_C, 32), jnp.float32),
            pltpu.VMEM((_AT_C, EMB), jnp.float32),
            pltpu.VMEM((_AT_C, 16), jnp.float32),
            pltpu.VMEM((_AT_C, 8), jnp.float32),
            pltpu.VMEM((_AT_C,), jnp.float32),
            pltpu.SemaphoreType.DMA,
        ],
        compiler_params=pltpu.CompilerParams(use_tc_tiling_on_sc=False, needs_layout_passes=False),
    )
    return f(kv48, q32, as32, ad32, ea20, src2, dst2, srcn2, dstn2)


def _seg_max_body(alpha_ref, dst_ref, amax_ref,
                  maxtab, a_v, d_v, sem):
    c = lax.axis_index("c")
    s = lax.axis_index("s")
    wid = s * 2 + c

    def initb(i, carry):
        maxtab[pl.ds(i * 16, 16)] = jnp.full((16,), -3e38, jnp.float32)
        return carry

    lax.fori_loop(0, _NP // 16, initb, 0)

    edge0 = wid * (_SM_CHUNKS * _SM_C)
    irow0 = edge0 // 128

    def chunk(ch, carry):
        base = edge0 + ch * _SM_C
        rb = irow0 + ch * (_SM_C // 128)
        pltpu.sync_copy(alpha_ref.at[pl.ds(base, _SM_C)], a_v)
        pltpu.sync_copy(dst_ref.at[pl.ds(rb, _SM_C // 128)], d_v)

        def block(g, carry2):
            a16 = a_v[pl.ds(g * 16, 16)]
            j = g // 8
            o = (g % 8) * 16
            d16 = d_v[j, pl.ds(o, 16)]

            def rnd(r, need):
                plsc.store_scatter(maxtab, [d16], a16, mask=need)
                cur = plsc.load_gather(maxtab, [d16])
                return jnp.logical_and(need, cur < a16)

            need0 = plsc.load_gather(maxtab, [d16]) < a16
            lax.fori_loop(0, 16, rnd, need0)
            return carry2

        lax.fori_loop(0, _SM_C // 16, block, 0)
        return carry

    lax.fori_loop(0, _SM_CHUNKS, chunk, 0)
    pltpu.sync_copy(maxtab, amax_ref.at[wid])


def _seg_max(alpha, dst2):
    mesh = plsc.VectorSubcoreMesh(core_axis_name="c", subcore_axis_name="s")
    f = pl.kernel(
        _seg_max_body,
        out_type=jax.ShapeDtypeStruct((32, _NP), jnp.float32),
        mesh=mesh,
        scratch_types=[
            pltpu.VMEM((_NP,), jnp.float32),
            pltpu.VMEM((_SM_C,), jnp.float32),
            pltpu.VMEM((_SM_C // 128, 128), jnp.int32),
            pltpu.SemaphoreType.DMA,
        ],
        compiler_params=pltpu.CompilerParams(use_tc_tiling_on_sc=False, needs_layout_passes=False),
    )
    return f(alpha, dst2)


def _amax_merge_body(p_ref, o_ref):
    o_ref[...] = jnp.max(p_ref[...], axis=0)


def _amax_merge(amax32):
    return pl.pallas_call(
        _amax_merge_body,
        out_shape=jax.ShapeDtypeStruct((_NP,), jnp.float32),
    )(amax32)


def _exp_den_body(alpha_ref, dst_ref, amax_ref, zeros_ref, al_ref, den_ref,
                  maxtab, a_v, d_v, al_v, den_sh, sem):
    c = lax.axis_index("c")
    s = lax.axis_index("s")
    wid = s * 2 + c
    r0 = s * _NPT
    pltpu.sync_copy(zeros_ref.at[pl.ds(r0, _NPT)], den_sh.at[pl.ds(r0, _NPT)])
    pltpu.sync_copy(amax_ref, maxtab)
    plsc.subcore_barrier()

    edge0 = wid * (_SM_CHUNKS * _SM_C)
    irow0 = edge0 // 128

    def chunk(ch, carry):
        base = edge0 + ch * _SM_C
        rb = irow0 + ch * (_SM_C // 128)
        pltpu.sync_copy(alpha_ref.at[pl.ds(base, _SM_C)], a_v)
        pltpu.sync_copy(dst_ref.at[pl.ds(rb, _SM_C // 128)], d_v)

        def block(g, carry2):
            a16 = a_v[pl.ds(g * 16, 16)]
            j = g // 8
            o = (g % 8) * 16
            d16 = d_v[j, pl.ds(o, 16)]
            mx16 = plsc.load_gather(maxtab, [d16])
            al_v[pl.ds(g * 16, 16)] = jnp.exp(a16 - mx16)
            return carry2

        lax.fori_loop(0, _SM_C // 16, block, 0)
        pltpu.sync_copy(al_v, al_ref.at[pl.ds(base, _SM_C)])
        for j in range(_SM_C // 128):
            pltpu.sync_copy(al_v.at[pl.ds(j * 128, 128)],
                            den_sh.at[d_v.at[j]], add=True)
        return carry

    lax.fori_loop(0, _SM_CHUNKS, chunk, 0)
    plsc.subcore_barrier()
    pltpu.sync_copy(den_sh.at[pl.ds(r0, _NPT)], den_ref.at[c, pl.ds(r0, _NPT)])


def _exp_den(alpha, dst2, amax, zerosN):
    mesh = plsc.VectorSubcoreMesh(core_axis_name="c", subcore_axis_name="s")
    f = pl.kernel(
        _exp_den_body,
        out_type=(jax.ShapeDtypeStruct((_EP,), jnp.float32),
                  jax.ShapeDtypeStruct((2, _NP), jnp.float32)),
        mesh=mesh,
        scratch_types=[
            pltpu.VMEM((_NP,), jnp.float32),
            pltpu.VMEM((_SM_C,), jnp.float32),
            pltpu.VMEM((_SM_C // 128, 128), jnp.int32),
            pltpu.VMEM((_SM_C,), jnp.float32),
            pltpu.VMEM_SHARED((_NP,), jnp.float32),
            pltpu.SemaphoreType.DMA,
        ],
        compiler_params=pltpu.CompilerParams(use_tc_tiling_on_sc=False, needs_layout_passes=False),
    )
    return f(alpha, dst2, amax, zerosN)


def _scatter_msg_body(w, msg_ref, al_ref, dst_ref, zeros_ref, out_ref,
                      m_v, al_v, d_v, w_v, acc, sem):
    c = lax.axis_index("c")
    s = lax.axis_index("s")
    wid = s * 2 + c
    r0 = s * _NPT
    pltpu.sync_copy(zeros_ref.at[pl.ds(r0, _NPT)], acc.at[pl.ds(r0, _NPT)])
    plsc.subcore_barrier()

    edge0 = wid * (_AT_CHUNKS * _AT_C)
    irow0 = edge0 // 128

    def chunk(ch, carry):
        base = edge0 + ch * _AT_C
        rb = irow0 + ch * (_AT_C // 128)
        pltpu.sync_copy(msg_ref.at[pl.ds(base, _AT_C)], m_v)
        pltpu.sync_copy(al_ref.at[pl.ds(base, _AT_C)], al_v)
        pltpu.sync_copy(dst_ref.at[pl.ds(rb, _AT_C // 128)], d_v)

        def block(g, carry2):
            i16 = _iota16() + g * 16
            w16 = al_v[pl.ds(g * 16, 16)]
            for d in range(w):
                cd = jnp.full((16,), d, jnp.int32)
                plsc.store_scatter(w_v, [i16, cd],
                                   plsc.load_gather(m_v, [i16, cd]) * w16)
            return carry2

        lax.fori_loop(0, _AT_C // 16, block, 0)
        for j in range(_AT_C // 128):
            pltpu.sync_copy(w_v.at[pl.ds(j * 128, 128)],
                            acc.at[d_v.at[j]], add=True)
        return carry

    lax.fori_loop(0, _AT_CHUNKS, chunk, 0)
    plsc.subcore_barrier()
    pltpu.sync_copy(acc.at[pl.ds(r0, _NPT)], out_ref.at[c, pl.ds(r0, _NPT)])


def _scatter_msg(msg, al, dst2, zerosNW, w):
    mesh = plsc.VectorSubcoreMesh(core_axis_name="c", subcore_axis_name="s")
    f = pl.kernel(
        functools.partial(_scatter_msg_body, w),
        out_type=jax.ShapeDtypeStruct((2, _NP, w), jnp.float32),
        mesh=mesh,
        scratch_types=[
            pltpu.VMEM((_AT_C, w), jnp.float32),
            pltpu.VMEM((_AT_C,), jnp.float32),
            pltpu.VMEM((_AT_C // 128, 128), jnp.int32),
            pltpu.VMEM((_AT_C, w), jnp.float32),
            pltpu.VMEM_SHARED((_NP, w), jnp.float32),
            pltpu.SemaphoreType.DMA,
        ],
        compiler_params=pltpu.CompilerParams(use_tc_tiling_on_sc=False, needs_layout_passes=False),
    )
    return f(msg, al, dst2, zerosNW)


def _final_body(pa_ref, pb_ref, den_ref, skip_ref, o_ref):
    a = jnp.concatenate(
        [pa_ref[0] + pa_ref[1], (pb_ref[0] + pb_ref[1])[:, :4]], axis=-1)
    dn = den_ref[:, 0] + den_ref[:, 1]
    safe = jnp.where(dn != 0.0, dn, 1.0)[:, None]
    o_ref[...] = jnp.where(dn[:, None] != 0.0, a / safe, 0.0) + skip_ref[...]


def _final(out_pa, out_pb, den_t, skip):
    blk = 2000
    return pl.pallas_call(
        _final_body,
        grid=(N // blk,),
        in_specs=[
            pl.BlockSpec((2, blk, 16), lambda i: (0, i, 0)),
            pl.BlockSpec((2, blk, 8), lambda i: (0, i, 0)),
            pl.BlockSpec((blk, 2), lambda i: (i, 0)),
            pl.BlockSpec((blk, EMB), lambda i: (i, 0)),
        ],
        out_specs=pl.BlockSpec((blk, EMB), lambda i: (i, 0)),
        out_shape=jax.ShapeDtypeStruct((N, EMB), jnp.float32),
    )(out_pa, out_pb, den_t, skip)


_NP_BLK = 3128


def _node_prep_body(x3f_ref, enc_ref, z_ref, W3T_ref, W1Tn_ref, W4T_ref,
                    W2Tn_ref, WqT_ref, WkT_ref, WvT_ref, WsT_ref, b_ref,
                    kv_ref, q_ref, as_ref, ad_ref, skip_ref):
    x3 = x3f_ref[:, :NODE_DIM]
    enc = enc_ref[...]
    z = z_ref[...]
    dot = functools.partial(jnp.dot, preferred_element_type=jnp.float32)
    As = dot(x3, W3T_ref[...]) + dot(enc, W1Tn_ref[...])
    Ad = dot(x3, W4T_ref[...]) + dot(enc, W2Tn_ref[...])
    q = dot(z, WqT_ref[...]) + b_ref[0:1, :]
    k = dot(z, WkT_ref[...]) + b_ref[1:2, :]
    v = dot(z, WvT_ref[...]) + b_ref[2:3, :]
    skip = dot(z, WsT_ref[...]) + b_ref[3:4, :]
    zpad = jnp.zeros((_NP_BLK, 12), jnp.float32)
    kv_ref[...] = jnp.concatenate([k, v, zpad[:, :8]], axis=-1)
    q_ref[...] = jnp.concatenate([q, zpad], axis=-1)
    as_ref[...] = jnp.concatenate([As, zpad], axis=-1)
    ad_ref[...] = jnp.concatenate([Ad, zpad], axis=-1)
    skip_ref[...] = skip


def _node_prep(x3f, encp, zp, W3T, W1Tn, W4T, W2Tn, WqT, WkT, WvT, WsT, b4):
    grid = (_NP // _NP_BLK,)
    w20 = lambda i: (0, 0)
    out = pl.pallas_call(
        _node_prep_body,
        grid=grid,
        in_specs=[
            pl.BlockSpec((_NP_BLK, 16), lambda i: (i, 0)),
            pl.BlockSpec((_NP_BLK, TIME), lambda i: (i, 0)),
            pl.BlockSpec((_NP_BLK, EMB), lambda i: (i, 0)),
            pl.BlockSpec((NODE_DIM, EMB), w20),
            pl.BlockSpec((TIME, EMB), w20),
            pl.BlockSpec((NODE_DIM, EMB), w20),
            pl.BlockSpec((TIME, EMB), w20),
            pl.BlockSpec((EMB, EMB), w20),
            pl.BlockSpec((EMB, EMB), w20),
            pl.BlockSpec((EMB, EMB), w20),
            pl.BlockSpec((EMB, EMB), w20),
            pl.BlockSpec((4, EMB), w20),
        ],
        out_specs=[
            pl.BlockSpec((_NP_BLK, 48), lambda i: (i, 0)),
            pl.BlockSpec((_NP_BLK, 32), lambda i: (i, 0)),
            pl.BlockSpec((_NP_BLK, 32), lambda i: (i, 0)),
            pl.BlockSpec((_NP_BLK, 32), lambda i: (i, 0)),
            pl.BlockSpec((_NP_BLK, EMB), lambda i: (i, 0)),
        ],
        out_shape=[
            jax.ShapeDtypeStruct((_NP, 48), jnp.float32),
            jax.ShapeDtypeStruct((_NP, 32), jnp.float32),
            jax.ShapeDtypeStruct((_NP, 32), jnp.float32),
            jax.ShapeDtypeStruct((_NP, 32), jnp.float32),
            jax.ShapeDtypeStruct((_NP, EMB), jnp.float32),
        ],
    )(x3f, encp, zp, W3T, W1Tn, W4T, W2Tn, WqT, WkT, WvT, WsT, b4)
    return out


def _edge_prep_body(ea_ref, t_ref, W0T_ref, s12_ref, out_ref):
    ea = ea_ref[...]
    t = t_ref[...]
    out_ref[...] = (
        jnp.dot(ea, W0T_ref[...], preferred_element_type=jnp.float32)
        + t * s12_ref[...]
    )


def _edge_prep(edge_attr_p, t_p, W0T, s12):
    blk = 8192
    grid = (_EP // blk,)
    out = pl.pallas_call(
        _edge_prep_body,
        grid=grid,
        in_specs=[
            pl.BlockSpec((blk, EDGE_DIM), lambda i: (i, 0)),
            pl.BlockSpec((blk, 1), lambda i: (i, 0)),
            pl.BlockSpec((EDGE_DIM, EMB), lambda i: (0, 0)),
            pl.BlockSpec((1, EMB), lambda i: (0, 0)),
        ],
        out_specs=pl.BlockSpec((blk, EMB), lambda i: (i, 0)),
        out_shape=jax.ShapeDtypeStruct((_EP, EMB), jnp.float32),
    )(edge_attr_p, t_p, W0T, s12)
    return out


def kernel(x, n_id, src_n_id, dst_n_id, edge_index, edge_attr, t,
           his_edge_index, enc_t_table, z,
           Wq, bq, Wk, bk, Wv, bv, We, Ws, bs):
    pad_idx = jnp.full((_MP - _M,), N, dtype=jnp.int32)
    src2 = jnp.concatenate(
        [edge_index[0], his_edge_index[0], pad_idx]).reshape(_MP // 128, 128)
    dst2 = jnp.concatenate(
        [edge_index[1], his_edge_index[1], pad_idx]).reshape(_MP // 128, 128)
    zeros16 = jnp.zeros((_NP, 16), jnp.float32)
    x16 = jnp.pad(x, ((0, _NP - N), (0, 16 - NODE_DIM)))
    for _ in range(3):
        x16 = _merge(_hop(x16, src2, dst2, zeros16))

    W0 = We[:, :EDGE_DIM]
    W1 = We[:, EDGE_DIM:EDGE_DIM + TIME]
    W2 = We[:, EDGE_DIM + TIME:EDGE_DIM + 2 * TIME]
    W3 = We[:, EDGE_DIM + 2 * TIME:EDGE_DIM + 2 * TIME + NODE_DIM]
    W4 = We[:, EDGE_DIM + 2 * TIME + NODE_DIM:]
    s12 = jnp.sum(W1 + W2, axis=1)[None, :]
    b4 = jnp.stack([bq, bk, bv, bs], axis=0)

    encp = jnp.pad(enc_t_table, ((0, _NP - N), (0, 0)))
    zp = jnp.pad(z, ((0, _NP - N), (0, 0)))
    kv48, q32, as32, ad32, skip = _node_prep(
        x16, encp, zp, W3.T, -W1.T, W4.T, -W2.T, Wq.T, Wk.T, Wv.T, Ws.T, b4)

    epad = _EP - E
    ea_p = jnp.pad(edge_attr, ((0, epad), (0, 0)))
    t_p = jnp.pad(t, (0, epad))[:, None]
    EA20 = _edge_prep(ea_p, t_p, W0.T, s12)

    e_pad_idx = jnp.full((epad,), N, dtype=jnp.int32)
    srcA = jnp.concatenate([edge_index[0], e_pad_idx]).reshape(_EP // 128, 128)
    dstA = jnp.concatenate([edge_index[1], e_pad_idx]).reshape(_EP // 128, 128)
    srcnA = jnp.concatenate([src_n_id, e_pad_idx]).reshape(_EP // 128, 128)
    dstnA = jnp.concatenate([dst_n_id, e_pad_idx]).reshape(_EP // 128, 128)

    msgA, msgB, alpha = _pass1(kv48, q32, as32, ad32, EA20,
                               srcA, dstA, srcnA, dstnA)
    amax = _amax_merge(_seg_max(alpha, dstA))
    zerosN = jnp.zeros((_NP,), jnp.float32)
    al, den_p = _exp_den(alpha, dstA, amax, zerosN)
    out_pa = _scatter_msg(msgA, al, dstA, zeros16, 16)
    zerosN8 = jnp.zeros((_NP, 8), jnp.float32)
    out_pb = _scatter_msg(msgB, al, dstA, zerosN8, 8)
    return _final(out_pa, out_pb, den_p.T, skip[:N])
    amax = _amax_merge(_seg_max(alpha, dstA))
    zerosN = jnp.zeros((_NP,), jnp.float32)
    al, den_p = _exp_den(alpha, dstA, amax, zerosN)
    out_pa = _scatter_msg(msgA, al, dstA, zeros16, 16)
    zerosN8 = jnp.zeros((_NP, 8), jnp.float32)
    out_pb = _scatter_msg(msgB, al, dstA, zerosN8, 8)
    return _final(out_pa, out_pb, den_p.T, skip[:N])


# speedup vs baseline: 28.2741x; 1.0379x over previous
import functools

import jax
import jax.numpy as jnp
from jax import lax
from jax.experimental import pallas as pl
from jax.experimental.pallas import tpu as pltpu
from jax.experimental.pallas import tpu_sc as plsc

N = 100000
NODE_DIM = 10
EDGE_DIM = 16
EMB = 20
TIME = 10
E = 1600000

_NODE_BLK = 2000
_EDGE_BLK = 8000

_NP = 100096
_M = 2 * E
_MP = 3276800
_HOP_IPB = 128
_HOP_K = 8
_HOP_CHUNKS = (_MP // 32) // (_HOP_K * _HOP_IPB)


def _hop_body(x_ref, src_ref, dst_ref, zeros_ref, out_ref,
              idxs_v, idxd_v, rows_v, acc, sem):
    c = lax.axis_index("c")
    s = lax.axis_index("s")
    wid = s * 2 + c
    rows_per_tile = _NP // 16
    r0 = s * rows_per_tile
    pltpu.sync_copy(zeros_ref.at[pl.ds(r0, rows_per_tile)],
                    acc.at[pl.ds(r0, rows_per_tile)])
    plsc.subcore_barrier()

    idx_row0 = wid * (_HOP_CHUNKS * _HOP_K)

    def chunk(ch, carry):
        rb = idx_row0 + ch * _HOP_K
        pltpu.sync_copy(src_ref.at[pl.ds(rb, _HOP_K)], idxs_v)
        cps = [pltpu.async_copy(x_ref.at[idxs_v.at[j]],
                                rows_v.at[pl.ds(j * _HOP_IPB, _HOP_IPB)], sem)
               for j in range(_HOP_K)]
        for cp in cps:
            cp.wait()
        pltpu.sync_copy(dst_ref.at[pl.ds(rb, _HOP_K)], idxd_v)
        for j in range(_HOP_K):
            pltpu.sync_copy(rows_v.at[pl.ds(j * _HOP_IPB, _HOP_IPB)],
                            acc.at[idxd_v.at[j]], add=True)
        return carry

    lax.fori_loop(0, _HOP_CHUNKS, chunk, 0)
    plsc.subcore_barrier()
    pltpu.sync_copy(acc.at[pl.ds(r0, rows_per_tile)],
                    out_ref.at[c, pl.ds(r0, rows_per_tile)])


def _hop(x16, src2, dst2, zeros16):
    mesh = plsc.VectorSubcoreMesh(core_axis_name="c", subcore_axis_name="s")
    f = pl.kernel(
        _hop_body,
        out_type=jax.ShapeDtypeStruct((2, _NP, 16), jnp.float32),
        mesh=mesh,
        scratch_types=[
            pltpu.VMEM((_HOP_K, _HOP_IPB), jnp.int32),
            pltpu.VMEM((_HOP_K, _HOP_IPB), jnp.int32),
            pltpu.VMEM((_HOP_K * _HOP_IPB, 16), jnp.float32),
            pltpu.VMEM_SHARED((_NP, 16), jnp.float32),
            pltpu.SemaphoreType.DMA,
        ],
        compiler_params=pltpu.CompilerParams(use_tc_tiling_on_sc=False, needs_layout_passes=False),
    )
    return f(x16, src2, dst2, zeros16)


def _merge_body(a_ref, b_ref, o_ref):
    o_ref[...] = a_ref[0] + b_ref[0]


def _merge(p):
    blk = 6256
    return pl.pallas_call(
        _merge_body,
        grid=(_NP // blk,),
        in_specs=[
            pl.BlockSpec((1, blk, 16), lambda i: (0, i, 0)),
            pl.BlockSpec((1, blk, 16), lambda i: (1, i, 0)),
        ],
        out_specs=pl.BlockSpec((blk, 16), lambda i: (i, 0)),
        out_shape=jax.ShapeDtypeStruct((_NP, 16), jnp.float32),
    )(p, p)


_EP = 1605632
_AT_C = 512
_AT_CHUNKS = (_EP // 32) // _AT_C
_SM_C = 1024
_SM_CHUNKS = (_EP // 32) // _SM_C
_NPT = _NP // 16
_ISQ = 0.22360679774997896


def _iota16():
    return lax.iota(jnp.int32, 16)


def _pass1_body(kv_ref, q_ref, as_ref, ad_ref, ea_ref,
                src_ref, dst_ref, srcn_ref, dstn_ref,
                msga_ref, msgb_ref, alpha_ref,
                si_v, di_v, ni_v, mi_v, kv_v, q_v, as_v, ad_v, ea_v,
                msga_v, msgb_v, al_v, sem):
    c = lax.axis_index("c")
    s = lax.axis_index("s")
    wid = s * 2 + c
    edge0 = wid * (_AT_CHUNKS * _AT_C)
    irow0 = edge0 // 128

    def chunk(ch, carry):
        base = edge0 + ch * _AT_C
        rb = irow0 + ch * (_AT_C // 128)
        nj = _AT_C // 128
        pltpu.sync_copy(src_ref.at[pl.ds(rb, nj)], si_v)
        pltpu.sync_copy(dst_ref.at[pl.ds(rb, nj)], di_v)
        pltpu.sync_copy(srcn_ref.at[pl.ds(rb, nj)], ni_v)
        pltpu.sync_copy(dstn_ref.at[pl.ds(rb, nj)], mi_v)
        cps = []
        for j in range(nj):
            sl = pl.ds(j * 128, 128)
            cps.append(pltpu.async_copy(kv_ref.at[si_v.at[j]], kv_v.at[sl], sem))
            cps.append(pltpu.async_copy(q_ref.at[di_v.at[j]], q_v.at[sl], sem))
            cps.append(pltpu.async_copy(as_ref.at[ni_v.at[j]], as_v.at[sl], sem))
            cps.append(pltpu.async_copy(ad_ref.at[mi_v.at[j]], ad_v.at[sl], sem))
        pltpu.sync_copy(ea_ref.at[pl.ds(base, _AT_C)], ea_v)
        for cp in cps:
            cp.wait()

        def block(g, carry2):
            i16 = _iota16() + g * 16
            acc = jnp.zeros((16,), jnp.float32)
            for d in range(EMB):
                cd = jnp.full((16,), d, jnp.int32)
                e_d = (plsc.load_gather(ea_v, [i16, cd])
                       + plsc.load_gather(as_v, [i16, cd])
                       + plsc.load_gather(ad_v, [i16, cd]))
                k_d = plsc.load_gather(kv_v, [i16, cd])
                v_d = plsc.load_gather(kv_v, [i16, jnp.full((16,), EMB + d, jnp.int32)])
                q_d = plsc.load_gather(q_v, [i16, cd])
                acc = acc + q_d * (k_d + e_d)
                if d < 16:
                    plsc.store_scatter(msga_v, [i16, cd], v_d + e_d)
                else:
                    cb = jnp.full((16,), d - 16, jnp.int32)
                    plsc.store_scatter(msgb_v, [i16, cb], v_d + e_d)
            z16 = jnp.zeros((16,), jnp.float32)
            for d in range(4, 8):
                plsc.store_scatter(msgb_v, [i16, jnp.full((16,), d, jnp.int32)], z16)
            al_v[pl.ds(g * 16, 16)] = acc * _ISQ
            return carry2

        lax.fori_loop(0, _AT_C // 16, block, 0)
        pltpu.sync_copy(msga_v, msga_ref.at[pl.ds(base, _AT_C)])
        pltpu.sync_copy(msgb_v, msgb_ref.at[pl.ds(base, _AT_C)])
        pltpu.sync_copy(al_v, alpha_ref.at[pl.ds(base, _AT_C)])
        return carry

    lax.fori_loop(0, _AT_CHUNKS, chunk, 0)


def _pass1(kv48, q32, as32, ad32, ea20, src2, dst2, srcn2, dstn2):
    mesh = plsc.VectorSubcoreMesh(core_axis_name="c", subcore_axis_name="s")
    f = pl.kernel(
        _pass1_body,
        out_type=(jax.ShapeDtypeStruct((_EP, 16), jnp.float32),
                  jax.ShapeDtypeStruct((_EP, 8), jnp.float32),
                  jax.ShapeDtypeStruct((_EP,), jnp.float32)),
        mesh=mesh,
        scratch_types=[
            pltpu.VMEM((_AT_C // 128, 128), jnp.int32),
            pltpu.VMEM((_AT_C // 128, 128), jnp.int32),
            pltpu.VMEM((_AT_C // 128, 128), jnp.int32),
            pltpu.VMEM((_AT_C // 128, 128), jnp.int32),
            pltpu.VMEM((_AT_C, 48), jnp.float32),
            pltpu.VMEM((_AT_C, 32), jnp.float32),
            pltpu.VMEM((_AT_C, 32), jnp.float32),
            pltpu.VMEM((_AT_C, 32), jnp.float32),
            pltpu.VMEM((_AT_C, EMB), jnp.float32),
            pltpu.VMEM((_AT_C, 16), jnp.float32),
            pltpu.VMEM((_AT_C, 8), jnp.float32),
            pltpu.VMEM((_AT_C,), jnp.float32),
            pltpu.SemaphoreType.DMA,
        ],
        compiler_params=pltpu.CompilerParams(use_tc_tiling_on_sc=False, needs_layout_passes=False),
    )
    return f(kv48, q32, as32, ad32, ea20, src2, dst2, srcn2, dstn2)


def _seg_max_body(alpha_ref, dst_ref, amax_ref,
                  maxtab, a_v, d_v, sem):
    c = lax.axis_index("c")
    s = lax.axis_index("s")
    wid = s * 2 + c

    def initb(i, carry):
        maxtab[pl.ds(i * 16, 16)] = jnp.full((16,), -3e38, jnp.float32)
        return carry

    lax.fori_loop(0, _NP // 16, initb, 0)

    edge0 = wid * (_SM_CHUNKS * _SM_C)
    irow0 = edge0 // 128

    def chunk(ch, carry):
        base = edge0 + ch * _SM_C
        rb = irow0 + ch * (_SM_C // 128)
        pltpu.sync_copy(alpha_ref.at[pl.ds(base, _SM_C)], a_v)
        pltpu.sync_copy(dst_ref.at[pl.ds(rb, _SM_C // 128)], d_v)

        def block(g, carry2):
            a16 = a_v[pl.ds(g * 16, 16)]
            j = g // 8
            o = (g % 8) * 16
            d16 = d_v[j, pl.ds(o, 16)]

            def rnd(r, need):
                plsc.store_scatter(maxtab, [d16], a16, mask=need)
                cur = plsc.load_gather(maxtab, [d16])
                return jnp.logical_and(need, cur < a16)

            need0 = plsc.load_gather(maxtab, [d16]) < a16
            lax.fori_loop(0, 16, rnd, need0)
            return carry2

        lax.fori_loop(0, _SM_C // 16, block, 0)
        return carry

    lax.fori_loop(0, _SM_CHUNKS, chunk, 0)
    pltpu.sync_copy(maxtab, amax_ref.at[wid])


def _seg_max(alpha, dst2):
    mesh = plsc.VectorSubcoreMesh(core_axis_name="c", subcore_axis_name="s")
    f = pl.kernel(
        _seg_max_body,
        out_type=jax.ShapeDtypeStruct((32, _NP), jnp.float32),
        mesh=mesh,
        scratch_types=[
            pltpu.VMEM((_NP,), jnp.float32),
            pltpu.VMEM((_SM_C,), jnp.float32),
            pltpu.VMEM((_SM_C // 128, 128), jnp.int32),
            pltpu.SemaphoreType.DMA,
        ],
        compiler_params=pltpu.CompilerParams(use_tc_tiling_on_sc=False, needs_layout_passes=False),
    )
    return f(alpha, dst2)


def _amax_merge_body(p_ref, o_ref):
    o_ref[...] = jnp.max(p_ref[...], axis=0)


def _amax_merge(amax32):
    return pl.pallas_call(
        _amax_merge_body,
        out_shape=jax.ShapeDtypeStruct((_NP,), jnp.float32),
    )(amax32)


def _exp_den_body(alpha_ref, dst_ref, amax_ref, zeros_ref, al_ref, den_ref,
                  maxtab, a_v, d_v, al_v, den_sh, sem):
    c = lax.axis_index("c")
    s = lax.axis_index("s")
    wid = s * 2 + c
    r0 = s * _NPT
    pltpu.sync_copy(zeros_ref.at[pl.ds(r0, _NPT)], den_sh.at[pl.ds(r0, _NPT)])
    pltpu.sync_copy(amax_ref, maxtab)
    plsc.subcore_barrier()

    edge0 = wid * (_SM_CHUNKS * _SM_C)
    irow0 = edge0 // 128

    def chunk(ch, carry):
        base = edge0 + ch * _SM_C
        rb = irow0 + ch * (_SM_C // 128)
        pltpu.sync_copy(alpha_ref.at[pl.ds(base, _SM_C)], a_v)
        pltpu.sync_copy(dst_ref.at[pl.ds(rb, _SM_C // 128)], d_v)

        def block(g, carry2):
            a16 = a_v[pl.ds(g * 16, 16)]
            j = g // 8
            o = (g % 8) * 16
            d16 = d_v[j, pl.ds(o, 16)]
            mx16 = plsc.load_gather(maxtab, [d16])
            al_v[pl.ds(g * 16, 16)] = jnp.exp(a16 - mx16)
            return carry2

        lax.fori_loop(0, _SM_C // 16, block, 0)
        pltpu.sync_copy(al_v, al_ref.at[pl.ds(base, _SM_C)])
        for j in range(_SM_C // 128):
            pltpu.sync_copy(al_v.at[pl.ds(j * 128, 128)],
                            den_sh.at[d_v.at[j]], add=True)
        return carry

    lax.fori_loop(0, _SM_CHUNKS, chunk, 0)
    plsc.subcore_barrier()
    pltpu.sync_copy(den_sh.at[pl.ds(r0, _NPT)], den_ref.at[c, pl.ds(r0, _NPT)])


def _exp_den(alpha, dst2, amax, zerosN):
    mesh = plsc.VectorSubcoreMesh(core_axis_name="c", subcore_axis_name="s")
    f = pl.kernel(
        _exp_den_body,
        out_type=(jax.ShapeDtypeStruct((_EP,), jnp.float32),
                  jax.ShapeDtypeStruct((2, _NP), jnp.float32)),
        mesh=mesh,
        scratch_types=[
            pltpu.VMEM((_NP,), jnp.float32),
            pltpu.VMEM((_SM_C,), jnp.float32),
            pltpu.VMEM((_SM_C // 128, 128), jnp.int32),
            pltpu.VMEM((_SM_C,), jnp.float32),
            pltpu.VMEM_SHARED((_NP,), jnp.float32),
            pltpu.SemaphoreType.DMA,
        ],
        compiler_params=pltpu.CompilerParams(use_tc_tiling_on_sc=False, needs_layout_passes=False),
    )
    return f(alpha, dst2, amax, zerosN)


def _scatter_msg_body(w, msg_ref, al_ref, dst_ref, zeros_ref, out_ref,
                      m_v, al_v, d_v, w_v, acc, sem):
    c = lax.axis_index("c")
    s = lax.axis_index("s")
    wid = s * 2 + c
    r0 = s * _NPT
    pltpu.sync_copy(zeros_ref.at[pl.ds(r0, _NPT)], acc.at[pl.ds(r0, _NPT)])
    plsc.subcore_barrier()

    edge0 = wid * (_AT_CHUNKS * _AT_C)
    irow0 = edge0 // 128

    def chunk(ch, carry):
        base = edge0 + ch * _AT_C
        rb = irow0 + ch * (_AT_C // 128)
        pltpu.sync_copy(msg_ref.at[pl.ds(base, _AT_C)], m_v)
        pltpu.sync_copy(al_ref.at[pl.ds(base, _AT_C)], al_v)
        pltpu.sync_copy(dst_ref.at[pl.ds(rb, _AT_C // 128)], d_v)

        def block(g, carry2):
            i16 = _iota16() + g * 16
            w16 = al_v[pl.ds(g * 16, 16)]
            for d in range(w):
                cd = jnp.full((16,), d, jnp.int32)
                plsc.store_scatter(w_v, [i16, cd],
                                   plsc.load_gather(m_v, [i16, cd]) * w16)
            return carry2

        lax.fori_loop(0, _AT_C // 16, block, 0)
        for j in range(_AT_C // 128):
            pltpu.sync_copy(w_v.at[pl.ds(j * 128, 128)],
                            acc.at[d_v.at[j]], add=True)
        return carry

    lax.fori_loop(0, _AT_CHUNKS, chunk, 0)
    plsc.subcore_barrier()
    pltpu.sync_copy(acc.at[pl.ds(r0, _NPT)], out_ref.at[c, pl.ds(r0, _NPT)])


def _scatter_msg(msg, al, dst2, zerosNW, w):
    mesh = plsc.VectorSubcoreMesh(core_axis_name="c", subcore_axis_name="s")
    f = pl.kernel(
        functools.partial(_scatter_msg_body, w),
        out_type=jax.ShapeDtypeStruct((2, _NP, w), jnp.float32),
        mesh=mesh,
        scratch_types=[
            pltpu.VMEM((_AT_C, w), jnp.float32),
            pltpu.VMEM((_AT_C,), jnp.float32),
            pltpu.VMEM((_AT_C // 128, 128), jnp.int32),
            pltpu.VMEM((_AT_C, w), jnp.float32),
            pltpu.VMEM_SHARED((_NP, w), jnp.float32),
            pltpu.SemaphoreType.DMA,
        ],
        compiler_params=pltpu.CompilerParams(use_tc_tiling_on_sc=False, needs_layout_passes=False),
    )
    return f(msg, al, dst2, zerosNW)


def _final_body(pa_ref, pb_ref, den_ref, skip_ref, o_ref):
    a = jnp.concatenate(
        [pa_ref[0] + pa_ref[1], (pb_ref[0] + pb_ref[1])[:, :4]], axis=-1)
    dn = den_ref[:, 0] + den_ref[:, 1]
    safe = jnp.where(dn != 0.0, dn, 1.0)[:, None]
    o_ref[...] = jnp.where(dn[:, None] != 0.0, a / safe, 0.0) + skip_ref[...]


def _final(out_pa, out_pb, den_t, skip):
    blk = 2000
    return pl.pallas_call(
        _final_body,
        grid=(N // blk,),
        in_specs=[
            pl.BlockSpec((2, blk, 16), lambda i: (0, i, 0)),
            pl.BlockSpec((2, blk, 8), lambda i: (0, i, 0)),
            pl.BlockSpec((blk, 2), lambda i: (i, 0)),
            pl.BlockSpec((blk, EMB), lambda i: (i, 0)),
        ],
        out_specs=pl.BlockSpec((blk, EMB), lambda i: (i, 0)),
        out_shape=jax.ShapeDtypeStruct((N, EMB), jnp.float32),
    )(out_pa, out_pb, den_t, skip)


_NP_BLK = 3128


def _node_prep_body(x3f_ref, enc_ref, z_ref, W3T_ref, W1Tn_ref, W4T_ref,
                    W2Tn_ref, WqT_ref, WkT_ref, WvT_ref, WsT_ref, b_ref,
                    kv_ref, q_ref, as_ref, ad_ref, skip_ref):
    x3 = x3f_ref[:, :NODE_DIM]
    enc = enc_ref[...]
    z = z_ref[...]
    dot = functools.partial(jnp.dot, preferred_element_type=jnp.float32)
    As = dot(x3, W3T_ref[...]) + dot(enc, W1Tn_ref[...])
    Ad = dot(x3, W4T_ref[...]) + dot(enc, W2Tn_ref[...])
    q = dot(z, WqT_ref[...]) + b_ref[0:1, :]
    k = dot(z, WkT_ref[...]) + b_ref[1:2, :]
    v = dot(z, WvT_ref[...]) + b_ref[2:3, :]
    skip = dot(z, WsT_ref[...]) + b_ref[3:4, :]
    zpad = jnp.zeros((_NP_BLK, 12), jnp.float32)
    kv_ref[...] = jnp.concatenate([k, v, zpad[:, :8]], axis=-1)
    q_ref[...] = jnp.concatenate([q, zpad], axis=-1)
    as_ref[...] = jnp.concatenate([As, zpad], axis=-1)
    ad_ref[...] = jnp.concatenate([Ad, zpad], axis=-1)
    skip_ref[...] = skip


def _node_prep(x3f, encp, zp, W3T, W1Tn, W4T, W2Tn, WqT, WkT, WvT, WsT, b4):
    grid = (_NP // _NP_BLK,)
    w20 = lambda i: (0, 0)
    out = pl.pallas_call(
        _node_prep_body,
        grid=grid,
        in_specs=[
            pl.BlockSpec((_NP_BLK, 16), lambda i: (i, 0)),
            pl.BlockSpec((_NP_BLK, TIME), lambda i: (i, 0)),
            pl.BlockSpec((_NP_BLK, EMB), lambda i: (i, 0)),
            pl.BlockSpec((NODE_DIM, EMB), w20),
            pl.BlockSpec((TIME, EMB), w20),
            pl.BlockSpec((NODE_DIM, EMB), w20),
            pl.BlockSpec((TIME, EMB), w20),
            pl.BlockSpec((EMB, EMB), w20),
            pl.BlockSpec((EMB, EMB), w20),
            pl.BlockSpec((EMB, EMB), w20),
            pl.BlockSpec((EMB, EMB), w20),
            pl.BlockSpec((4, EMB), w20),
        ],
        out_specs=[
            pl.BlockSpec((_NP_BLK, 48), lambda i: (i, 0)),
            pl.BlockSpec((_NP_BLK, 32), lambda i: (i, 0)),
            pl.BlockSpec((_NP_BLK, 32), lambda i: (i, 0)),
            pl.BlockSpec((_NP_BLK, 32), lambda i: (i, 0)),
            pl.BlockSpec((_NP_BLK, EMB), lambda i: (i, 0)),
        ],
        out_shape=[
            jax.ShapeDtypeStruct((_NP, 48), jnp.float32),
            jax.ShapeDtypeStruct((_NP, 32), jnp.float32),
            jax.ShapeDtypeStruct((_NP, 32), jnp.float32),
            jax.ShapeDtypeStruct((_NP, 32), jnp.float32),
            jax.ShapeDtypeStruct((_NP, EMB), jnp.float32),
        ],
    )(x3f, encp, zp, W3T, W1Tn, W4T, W2Tn, WqT, WkT, WvT, WsT, b4)
    return out


def _edge_prep_body(ea_ref, t_ref, W0T_ref, s12_ref, out_ref):
    ea = ea_ref[...]
    t = t_ref[...]
    out_ref[...] = (
        jnp.dot(ea, W0T_ref[...], preferred_element_type=jnp.float32)
        + t * s12_ref[...]
    )


def _edge_prep(edge_attr_p, t_p, W0T, s12):
    blk = 8192
    grid = (_EP // blk,)
    out = pl.pallas_call(
        _edge_prep_body,
        grid=grid,
        in_specs=[
            pl.BlockSpec((blk, EDGE_DIM), lambda i: (i, 0)),
            pl.BlockSpec((blk, 1), lambda i: (i, 0)),
            pl.BlockSpec((EDGE_DIM, EMB), lambda i: (0, 0)),
            pl.BlockSpec((1, EMB), lambda i: (0, 0)),
        ],
        out_specs=pl.BlockSpec((blk, EMB), lambda i: (i, 0)),
        out_shape=jax.ShapeDtypeStruct((_EP, EMB), jnp.float32),
    )(edge_attr_p, t_p, W0T, s12)
    return out


def kernel(x, n_id, src_n_id, dst_n_id, edge_index, edge_attr, t,
           his_edge_index, enc_t_table, z,
           Wq, bq, Wk, bk, Wv, bv, We, Ws, bs):
    pad_idx = jnp.full((_MP - _M,), N, dtype=jnp.int32)
    src2 = jnp.concatenate(
        [edge_index[0], his_edge_index[0], pad_idx]).reshape(_MP // 128, 128)
    dst2 = jnp.concatenate(
        [edge_index[1], his_edge_index[1], pad_idx]).reshape(_MP // 128, 128)
    zeros16 = jnp.zeros((_NP, 16), jnp.float32)
    x16 = jnp.pad(x, ((0, _NP - N), (0, 16 - NODE_DIM)))
    for _ in range(3):
        x16 = _merge(_hop(x16, src2, dst2, zeros16))

    W0 = We[:, :EDGE_DIM]
    W1 = We[:, EDGE_DIM:EDGE_DIM + TIME]
    W2 = We[:, EDGE_DIM + TIME:EDGE_DIM + 2 * TIME]
    W3 = We[:, EDGE_DIM + 2 * TIME:EDGE_DIM + 2 * TIME + NODE_DIM]
    W4 = We[:, EDGE_DIM + 2 * TIME + NODE_DIM:]
    s12 = jnp.sum(W1 + W2, axis=1)[None, :]
    b4 = jnp.stack([bq, bk, bv, bs], axis=0)

    encp = jnp.pad(enc_t_table, ((0, _NP - N), (0, 0)))
    zp = jnp.pad(z, ((0, _NP - N), (0, 0)))
    kv48, q32, as32, ad32, skip = _node_prep(
        x16, encp, zp, W3.T, -W1.T, W4.T, -W2.T, Wq.T, Wk.T, Wv.T, Ws.T, b4)

    epad = _EP - E
    ea_p = jnp.pad(edge_attr, ((0, epad), (0, 0)))
    t_p = jnp.pad(t, (0, epad))[:, None]
    EA20 = _edge_prep(ea_p, t_p, W0.T, s12)

    e_pad_idx = jnp.full((epad,), N, dtype=jnp.int32)
    srcA = jnp.concatenate([edge_index[0], e_pad_idx]).reshape(_EP // 128, 128)
    dstA = jnp.concatenate([edge_index[1], e_pad_idx]).reshape(_EP // 128, 128)
    srcnA = jnp.concatenate([src_n_id, e_pad_idx]).reshape(_EP // 128, 128)
    dstnA = jnp.concatenate([dst_n_id, e_pad_idx]).reshape(_EP // 128, 128)

    msgA, msgB, alpha = _pass1(kv48, q32, as32, ad32, EA20,
                               srcA, dstA, srcnA, dstnA)
    amax = _amax_merge(_seg_max(alpha, dstA))
    zerosN = jnp.zeros((_NP,), jnp.float32)
    al, den_p = _exp_den(alpha, dstA, amax, zerosN)
    out_pa = _scatter_msg(msgA, al, dstA, zeros16, 16)
    zerosN8 = jnp.zeros((_NP, 8), jnp.float32)
    out_pb = _scatter_msg(msgB, al, dstA, zerosN8, 8)
    return _final(out_pa, out_pb, den_p.T, skip[:N])
    amax = _amax_merge(_seg_max(alpha, dstA))
    zerosN = jnp.zeros((_NP,), jnp.float32)
    al, den_p = _exp_den(alpha, dstA, amax, zerosN)
    out_pa = _scatter_msg(msgA, al, dstA, zeros16, 16)
    zerosN8 = jnp.zeros((_NP, 8), jnp.float32)
    out_pb = _scatter_msg(msgB, al, dstA, zerosN8, 8)
    return _final(out_pa, out_pb, den_p.T, skip[:N])
